# baseline (speedup 1.0000x reference)
import functools

import jax
import jax.numpy as jnp
from jax import lax
from jax.experimental import pallas as pl
from jax.experimental.pallas import tpu as pltpu
from jax.experimental.pallas import tpu_sc as plsc

N = 10000
E = 320000
D = 8
NG = 64
STEPS = 3

NC = 2
NS = 16
NW = NC * NS
EPW = E // NW
CW = 125
SUB = 8
SCW = CW * SUB
NSC = EPW // SCW
NCH = EPW // CW
ROWS_PER_TILE = N // NS

_SC_PARAMS = pltpu.CompilerParams(use_tc_tiling_on_sc=False)


@functools.cache
def _sc_mesh():
    return plsc.VectorSubcoreMesh(
        core_axis_name="c", subcore_axis_name="s", num_cores=NC, num_subcores=NS
    )


def _gather_body(h_hbm, src_hbm, out_hbm, idx_v, rows_v, gsem, ssem):
    wid = lax.axis_index("s") * NC + lax.axis_index("c")
    base = wid * EPW
    pltpu.sync_copy(src_hbm.at[wid], idx_v)

    def gathers(m, half):
        descs = []
        for b in range(SUB):
            descs.append(pltpu.async_copy(
                h_hbm.at[idx_v.at[m * SUB + b]],
                rows_v.at[half, pl.ds(b * CW, CW)],
                gsem,
            ))
        return descs

    def store_desc(m, half):
        return pltpu.make_async_copy(
            rows_v.at[half], out_hbm.at[pl.ds(base + m * SCW, SCW)], ssem
        )

    def body(m, carry):
        half = lax.rem(m, 2)

        @pl.when(m >= 2)
        def _():
            store_desc(m - 2, half).wait()

        descs = gathers(m, half)
        for dsc in descs:
            dsc.wait()
        pltpu.async_copy(
            rows_v.at[half], out_hbm.at[pl.ds(base + m * SCW, SCW)], ssem
        )
        return carry

    lax.fori_loop(0, NSC, body, 0)
    store_desc(NSC - 2, lax.rem(NSC - 2, 2)).wait()
    store_desc(NSC - 1, lax.rem(NSC - 1, 2)).wait()


def _sc_gather(h, src3):
    kern = pl.kernel(
        _gather_body,
        out_type=jax.ShapeDtypeStruct((E, D), jnp.float32),
        mesh=_sc_mesh(),
        scratch_types=[
            pltpu.VMEM((NCH, CW), jnp.int32),
            pltpu.VMEM((2, SCW, D), jnp.float32),
            pltpu.SemaphoreType.DMA,
            pltpu.SemaphoreType.DMA,
        ],
        compiler_params=_SC_PARAMS,
    )
    return kern(h, src3)


def _scatter_body(msg_hbm, dst_hbm, zeros_hbm, out_hbm, didx_v, mrows_v, agg_sh,
                  lsem, asem):
    cid = lax.axis_index("c")
    sid = lax.axis_index("s")
    wid = sid * NC + cid
    base = wid * EPW

    pltpu.sync_copy(
        zeros_hbm.at[pl.ds(sid * ROWS_PER_TILE, ROWS_PER_TILE)],
        agg_sh.at[pl.ds(sid * ROWS_PER_TILE, ROWS_PER_TILE)],
    )
    pltpu.sync_copy(dst_hbm.at[wid], didx_v)
    plsc.subcore_barrier()

    def load_desc(m, half):
        return pltpu.make_async_copy(
            msg_hbm.at[pl.ds(base + m * SCW, SCW)], mrows_v.at[half], lsem
        )

    def scat_desc(m, half, b):
        return pltpu.make_async_copy(
            mrows_v.at[half, pl.ds(b * CW, CW)],
            agg_sh.at[didx_v.at[m * SUB + b]],
            asem,
        )

    def body(m, carry):
        half = lax.rem(m, 2)

        @pl.when(m >= 2)
        def _():
            for b in range(SUB):
                scat_desc(m - 2, half, b).wait()

        load_desc(m, half).start()
        load_desc(m, half).wait()
        for b in range(SUB):
            pltpu.async_copy(
                mrows_v.at[half, pl.ds(b * CW, CW)],
                agg_sh.at[didx_v.at[m * SUB + b]],
                asem,
                add=True,
            )
        return carry

    lax.fori_loop(0, NSC, body, 0)
    for m in (NSC - 2, NSC - 1):
        for b in range(SUB):
            scat_desc(m, m % 2, b).wait()
    plsc.subcore_barrier()

    pltpu.sync_copy(
        agg_sh.at[pl.ds(sid * ROWS_PER_TILE, ROWS_PER_TILE)],
        out_hbm.at[cid, pl.ds(sid * ROWS_PER_TILE, ROWS_PER_TILE)],
    )


def _sc_scatter_add(msg, dst3, zeros_nd):
    kern = pl.kernel(
        _scatter_body,
        out_type=jax.ShapeDtypeStruct((NC, N, D), jnp.float32),
        mesh=_sc_mesh(),
        scratch_types=[
            pltpu.VMEM((NCH, CW), jnp.int32),
            pltpu.VMEM((2, SCW, D), jnp.float32),
            pltpu.VMEM_SHARED((N, D), jnp.float32),
            pltpu.SemaphoreType.DMA,
            pltpu.SemaphoreType.DMA,
        ],
        compiler_params=_SC_PARAMS,
    )
    return kern(msg, dst3, zeros_nd)


def _proj_body(x_ref, w_ref, b_ref, o_ref):
    o_ref[...] = jnp.maximum(
        jnp.dot(x_ref[...], w_ref[...], preferred_element_type=jnp.float32)
        + b_ref[...],
        0.0,
    )


def _tc_project(x, W_proj, b_proj):
    return pl.pallas_call(
        _proj_body,
        out_shape=jax.ShapeDtypeStruct((N, D), jnp.float32),
    )(x, W_proj, b_proj.reshape(1, D))


MSG_BM = 6400


def _msg_body(eat_ref, hsp_ref, we1_ref, be1_ref, we2_ref, be2_ref, r_ref, s_ref, o_ref):
    eh = jnp.maximum(
        lax.dot_general(
            eat_ref[...], we1_ref[...], (((0,), (0,)), ((), ())),
            preferred_element_type=jnp.float32,
        )
        + be1_ref[...],
        0.0,
    )
    ew = (
        jnp.dot(eh, we2_ref[...], preferred_element_type=jnp.float32)
        + be2_ref[...]
    )
    hsp = hsp_ref[...]
    hs = jnp.stack(
        [hsp[:, D * k:D * (k + 1)] for k in range(16)], axis=1
    ).reshape(MSG_BM, D)
    hr = jnp.dot(hs, r_ref[...], preferred_element_type=jnp.float32)
    msg = jnp.dot(ew * hr, s_ref[...], preferred_element_type=jnp.float32)
    msg3 = msg.reshape(MSG_BM // 16, 16, D)
    o_ref[...] = jnp.concatenate([msg3[:, k, :] for k in range(16)], axis=1)


def _tc_message(eaT, h_src_p, W_e1, b_e1, W_e2, b_e2, R, S):
    grid = E // MSG_BM
    pb = MSG_BM * D // 128
    return pl.pallas_call(
        _msg_body,
        grid=(grid,),
        in_specs=[
            pl.BlockSpec((16, MSG_BM), lambda i: (0, i)),
            pl.BlockSpec((pb, 128), lambda i: (i, 0)),
            pl.BlockSpec((16, 16), lambda i: (0, 0)),
            pl.BlockSpec((1, 16), lambda i: (0, 0)),
            pl.BlockSpec((16, D * D), lambda i: (0, 0)),
            pl.BlockSpec((1, D * D), lambda i: (0, 0)),
            pl.BlockSpec((D, D * D), lambda i: (0, 0)),
            pl.BlockSpec((D * D, D), lambda i: (0, 0)),
        ],
        out_specs=pl.BlockSpec((pb, 128), lambda i: (i, 0)),
        out_shape=jax.ShapeDtypeStruct((E * D // 128, 128), jnp.float32),
    )(eaT, h_src_p, W_e1, b_e1.reshape(1, 16), W_e2, b_e2.reshape(1, D * D), R, S)


def _gru_body(
    agg2_ref, h_ref, hid_ref, wroot_ref, bconv_ref,
    wir_ref, wiz_ref, win_ref, bir_ref, biz_ref, bin_ref,
    whr_ref, whz_ref, whn_ref, bhr_ref, bhz_ref, bhn_ref,
    o_ref,
):
    agg = agg2_ref[0] + agg2_ref[1]
    h = h_ref[...]
    hidden = hid_ref[...]
    m = jnp.maximum(
        agg
        + jnp.dot(h, wroot_ref[...], preferred_element_type=jnp.float32)
        + bconv_ref[...],
        0.0,
    )
    i_r = jnp.dot(m, wir_ref[...], preferred_element_type=jnp.float32) + bir_ref[...]
    i_z = jnp.dot(m, wiz_ref[...], preferred_element_type=jnp.float32) + biz_ref[...]
    i_n = jnp.dot(m, win_ref[...], preferred_element_type=jnp.float32) + bin_ref[...]
    h_r = jnp.dot(hidden, whr_ref[...], preferred_element_type=jnp.float32) + bhr_ref[...]
    h_z = jnp.dot(hidden, whz_ref[...], preferred_element_type=jnp.float32) + bhz_ref[...]
    h_n = jnp.dot(hidden, whn_ref[...], preferred_element_type=jnp.float32) + bhn_ref[...]
    r = jax.nn.sigmoid(i_r + h_r)
    z = jax.nn.sigmoid(i_z + h_z)
    n = jnp.tanh(i_n + r * h_n)
    o_ref[...] = (1.0 - z) * n + z * hidden


def _tc_gru(agg2, h, hidden, W_root, b_conv, gru_w):
    (wir, wiz, win, bir, biz, bin_, whr, whz, whn, bhr, bhz, bhn) = gru_w
    return pl.pallas_call(
        _gru_body,
        out_shape=jax.ShapeDtypeStruct((N, D), jnp.float32),
    )(agg2, h, hidden, W_root, b_conv.reshape(1, D),
      wir, wiz, win, bir, biz, bin_, whr, whz, whn, bhr, bhz, bhn)


def _readout_body(
    h_ref, batch_ref, wr1_ref, br1_ref, wr2_ref, br2_ref, wp_ref, bp_ref, o_ref
):
    h = h_ref[...]
    nf = jnp.maximum(
        jnp.dot(h, wr1_ref[...], preferred_element_type=jnp.float32) + br1_ref[...],
        0.0,
    )
    nf = jnp.dot(nf, wr2_ref[...], preferred_element_type=jnp.float32) + br2_ref[...]
    gid = lax.broadcasted_iota(jnp.int32, (1, NG), 1)
    oh = (batch_ref[...] == gid).astype(jnp.float32)
    sums = lax.dot_general(
        oh, nf, (((0,), (0,)), ((), ())), preferred_element_type=jnp.float32
    )
    counts = lax.dot_general(
        oh,
        jnp.ones((N, 1), jnp.float32),
        (((0,), (0,)), ((), ())),
        preferred_element_type=jnp.float32,
    )
    g = sums / jnp.maximum(counts, 1.0)
    o_ref[...] = (
        jnp.dot(g, wp_ref[...], preferred_element_type=jnp.float32) + bp_ref[...]
    )


def _tc_readout(h, batch2d, W_r1, b_r1, W_r2, b_r2, W_p, b_p):
    return pl.pallas_call(
        _readout_body,
        out_shape=jax.ShapeDtypeStruct((NG, 1), jnp.float32),
    )(h, batch2d, W_r1, b_r1.reshape(1, D), W_r2, b_r2.reshape(1, D),
      W_p, b_p.reshape(1, 1))


def kernel(x, edge_index, edge_attr, batch,
           W_proj, b_proj, W_e1, b_e1, W_e2, b_e2, W_root, b_conv,
           W_gru_ih, b_gru_ih, W_gru_hh, b_gru_hh,
           W_r1, b_r1, W_r2, b_r2, W_p, b_p):
    src3 = edge_index[0].reshape(NW, NCH, CW)
    dst3 = edge_index[1].reshape(NW, NCH, CW)
    batch2d = batch.reshape(N, 1)
    eaT = edge_attr.T
    zeros_nd = jnp.zeros((N, D), jnp.float32)

    i8 = jnp.arange(D)
    i64 = jnp.arange(D * D)
    R = (i64[None, :] // D == i8[:, None]).astype(jnp.float32)
    S = (i64[:, None] % D == i8[None, :]).astype(jnp.float32)

    gru_w = (
        W_gru_ih[:, 0:D], W_gru_ih[:, D:2 * D], W_gru_ih[:, 2 * D:3 * D],
        b_gru_ih[0:D].reshape(1, D), b_gru_ih[D:2 * D].reshape(1, D),
        b_gru_ih[2 * D:3 * D].reshape(1, D),
        W_gru_hh[:, 0:D], W_gru_hh[:, D:2 * D], W_gru_hh[:, 2 * D:3 * D],
        b_gru_hh[0:D].reshape(1, D), b_gru_hh[D:2 * D].reshape(1, D),
        b_gru_hh[2 * D:3 * D].reshape(1, D),
    )

    h = _tc_project(x, W_proj, b_proj)
    hidden = h
    for _ in range(STEPS):
        h_src = _sc_gather(h, src3)
        msg_p = _tc_message(eaT, h_src.reshape(E * D // 128, 128),
                            W_e1, b_e1, W_e2, b_e2, R, S)
        agg2 = _sc_scatter_add(msg_p.reshape(E, D), dst3, zeros_nd)
        hidden = _tc_gru(agg2, h, hidden, W_root, b_conv, gru_w)
        h = hidden
    return _tc_readout(h, batch2d, W_r1, b_r1, W_r2, b_r2, W_p, b_p)


# speedup vs baseline: 3.7714x; 3.7714x over previous
import functools

import jax
import jax.numpy as jnp
from jax import lax
from jax.experimental import pallas as pl
from jax.experimental.pallas import tpu as pltpu
from jax.experimental.pallas import tpu_sc as plsc

N = 10000
E = 320000
D = 8
NG = 64
STEPS = 3

NC = 2
NS = 16
NW = NC * NS
EPW = E // NW
CW = 125
SUB = 8
SCW = CW * SUB
NSC = EPW // SCW
NCH = EPW // CW
ROWS_PER_TILE = N // NS

_SC_PARAMS = pltpu.CompilerParams(use_tc_tiling_on_sc=False)


@functools.cache
def _sc_mesh():
    return plsc.VectorSubcoreMesh(
        core_axis_name="c", subcore_axis_name="s", num_cores=NC, num_subcores=NS
    )


def _gather_body(h_hbm, src_hbm, out_hbm, idx_v, rows_v, gsem, ssem):
    wid = lax.axis_index("s") * NC + lax.axis_index("c")
    base = wid * EPW
    pltpu.sync_copy(src_hbm.at[wid], idx_v)

    def gathers(m, half):
        descs = []
        for b in range(SUB):
            descs.append(pltpu.async_copy(
                h_hbm.at[idx_v.at[m * SUB + b]],
                rows_v.at[half, pl.ds(b * CW, CW)],
                gsem,
            ))
        return descs

    def store_desc(m, half):
        return pltpu.make_async_copy(
            rows_v.at[half], out_hbm.at[pl.ds(base + m * SCW, SCW)], ssem
        )

    def body(m, carry):
        half = lax.rem(m, 2)

        @pl.when(m >= 2)
        def _():
            store_desc(m - 2, half).wait()

        descs = gathers(m, half)
        for dsc in descs:
            dsc.wait()
        pltpu.async_copy(
            rows_v.at[half], out_hbm.at[pl.ds(base + m * SCW, SCW)], ssem
        )
        return carry

    lax.fori_loop(0, NSC, body, 0)
    store_desc(NSC - 2, lax.rem(NSC - 2, 2)).wait()
    store_desc(NSC - 1, lax.rem(NSC - 1, 2)).wait()


def _sc_gather(h, src3):
    kern = pl.kernel(
        _gather_body,
        out_type=jax.ShapeDtypeStruct((E, D), jnp.float32),
        mesh=_sc_mesh(),
        scratch_types=[
            pltpu.VMEM((NCH, CW), jnp.int32),
            pltpu.VMEM((2, SCW, D), jnp.float32),
            pltpu.SemaphoreType.DMA,
            pltpu.SemaphoreType.DMA,
        ],
        compiler_params=_SC_PARAMS,
    )
    return kern(h, src3)


def _scatter_body(msg_hbm, dst_hbm, zeros_hbm, out_hbm, didx_v, mrows_v, agg_sh,
                  lsem, asem):
    cid = lax.axis_index("c")
    sid = lax.axis_index("s")
    wid = sid * NC + cid
    base = wid * EPW

    pltpu.sync_copy(
        zeros_hbm.at[pl.ds(sid * ROWS_PER_TILE, ROWS_PER_TILE)],
        agg_sh.at[pl.ds(sid * ROWS_PER_TILE, ROWS_PER_TILE)],
    )
    pltpu.sync_copy(dst_hbm.at[wid], didx_v)
    plsc.subcore_barrier()

    def load_desc(m, half):
        return pltpu.make_async_copy(
            msg_hbm.at[pl.ds(base + m * SCW, SCW)], mrows_v.at[half], lsem
        )

    def scat_desc(m, half, b):
        return pltpu.make_async_copy(
            mrows_v.at[half, pl.ds(b * CW, CW)],
            agg_sh.at[didx_v.at[m * SUB + b]],
            asem,
        )

    def body(m, carry):
        half = lax.rem(m, 2)

        @pl.when(m >= 2)
        def _():
            for b in range(SUB):
                scat_desc(m - 2, half, b).wait()

        load_desc(m, half).start()
        load_desc(m, half).wait()
        for b in range(SUB):
            pltpu.async_copy(
                mrows_v.at[half, pl.ds(b * CW, CW)],
                agg_sh.at[didx_v.at[m * SUB + b]],
                asem,
                add=True,
            )
        return carry

    lax.fori_loop(0, NSC, body, 0)
    for m in (NSC - 2, NSC - 1):
        for b in range(SUB):
            scat_desc(m, m % 2, b).wait()
    plsc.subcore_barrier()

    pltpu.sync_copy(
        agg_sh.at[pl.ds(sid * ROWS_PER_TILE, ROWS_PER_TILE)],
        out_hbm.at[cid, pl.ds(sid * ROWS_PER_TILE, ROWS_PER_TILE)],
    )


def _sc_scatter_add(msg, dst3, zeros_nd):
    kern = pl.kernel(
        _scatter_body,
        out_type=jax.ShapeDtypeStruct((NC, N, D), jnp.float32),
        mesh=_sc_mesh(),
        scratch_types=[
            pltpu.VMEM((NCH, CW), jnp.int32),
            pltpu.VMEM((2, SCW, D), jnp.float32),
            pltpu.VMEM_SHARED((N, D), jnp.float32),
            pltpu.SemaphoreType.DMA,
            pltpu.SemaphoreType.DMA,
        ],
        compiler_params=_SC_PARAMS,
    )
    return kern(msg, dst3, zeros_nd)


def _proj_body(x_ref, w_ref, b_ref, o_ref):
    o_ref[...] = jnp.maximum(
        jnp.dot(x_ref[...], w_ref[...], preferred_element_type=jnp.float32)
        + b_ref[...],
        0.0,
    )


def _tc_project(x, W_proj, b_proj):
    return pl.pallas_call(
        _proj_body,
        out_shape=jax.ShapeDtypeStruct((N, D), jnp.float32),
    )(x, W_proj, b_proj.reshape(1, D))


MSG_BM = 16000


def _msg_body(eap_ref, hsp_ref, bd1_ref, b1t_ref, bd2_ref, b2t_ref, g_ref, o_ref):
    ehp = jnp.maximum(
        jnp.dot(eap_ref[...], bd1_ref[...], preferred_element_type=jnp.float32)
        + b1t_ref[...],
        0.0,
    )
    ew2 = jnp.dot(ehp, bd2_ref[...], preferred_element_type=jnp.float32)
    hs2 = jnp.dot(hsp_ref[...], g_ref[...], preferred_element_type=jnp.float32)
    b2t = b2t_ref[...]
    acc = hs2[:, 0:128] * (ew2[:, 0:128] + b2t[0:1, :])
    for i in range(1, D):
        acc += hs2[:, 128 * i:128 * (i + 1)] * (
            ew2[:, 128 * i:128 * (i + 1)] + b2t[i:i + 1, :]
        )
    o_ref[...] = acc


def _tc_message(eap, h_src_p, BD1, b1t, BD2, b2t, G):
    grid = E // MSG_BM
    pb = MSG_BM // 16
    return pl.pallas_call(
        _msg_body,
        grid=(grid,),
        in_specs=[
            pl.BlockSpec((pb, 256), lambda i: (i, 0)),
            pl.BlockSpec((pb, 128), lambda i: (i, 0)),
            pl.BlockSpec((256, 256), lambda i: (0, 0)),
            pl.BlockSpec((1, 256), lambda i: (0, 0)),
            pl.BlockSpec((256, D * 128), lambda i: (0, 0)),
            pl.BlockSpec((D, 128), lambda i: (0, 0)),
            pl.BlockSpec((128, D * 128), lambda i: (0, 0)),
        ],
        out_specs=pl.BlockSpec((pb, 128), lambda i: (i, 0)),
        out_shape=jax.ShapeDtypeStruct((E // 16, 128), jnp.float32),
    )(eap, h_src_p, BD1, b1t, BD2, b2t, G)


def _gru_body(
    agg2_ref, h_ref, hid_ref, wroot_ref, bconv_ref,
    wir_ref, wiz_ref, win_ref, bir_ref, biz_ref, bin_ref,
    whr_ref, whz_ref, whn_ref, bhr_ref, bhz_ref, bhn_ref,
    o_ref,
):
    agg = agg2_ref[0] + agg2_ref[1]
    h = h_ref[...]
    hidden = hid_ref[...]
    m = jnp.maximum(
        agg
        + jnp.dot(h, wroot_ref[...], preferred_element_type=jnp.float32)
        + bconv_ref[...],
        0.0,
    )
    i_r = jnp.dot(m, wir_ref[...], preferred_element_type=jnp.float32) + bir_ref[...]
    i_z = jnp.dot(m, wiz_ref[...], preferred_element_type=jnp.float32) + biz_ref[...]
    i_n = jnp.dot(m, win_ref[...], preferred_element_type=jnp.float32) + bin_ref[...]
    h_r = jnp.dot(hidden, whr_ref[...], preferred_element_type=jnp.float32) + bhr_ref[...]
    h_z = jnp.dot(hidden, whz_ref[...], preferred_element_type=jnp.float32) + bhz_ref[...]
    h_n = jnp.dot(hidden, whn_ref[...], preferred_element_type=jnp.float32) + bhn_ref[...]
    r = jax.nn.sigmoid(i_r + h_r)
    z = jax.nn.sigmoid(i_z + h_z)
    n = jnp.tanh(i_n + r * h_n)
    o_ref[...] = (1.0 - z) * n + z * hidden


def _tc_gru(agg2, h, hidden, W_root, b_conv, gru_w):
    (wir, wiz, win, bir, biz, bin_, whr, whz, whn, bhr, bhz, bhn) = gru_w
    return pl.pallas_call(
        _gru_body,
        out_shape=jax.ShapeDtypeStruct((N, D), jnp.float32),
    )(agg2, h, hidden, W_root, b_conv.reshape(1, D),
      wir, wiz, win, bir, biz, bin_, whr, whz, whn, bhr, bhz, bhn)


def _readout_body(
    h_ref, batch_ref, wr1_ref, br1_ref, wr2_ref, br2_ref, wp_ref, bp_ref, o_ref
):
    h = h_ref[...]
    nf = jnp.maximum(
        jnp.dot(h, wr1_ref[...], preferred_element_type=jnp.float32) + br1_ref[...],
        0.0,
    )
    nf = jnp.dot(nf, wr2_ref[...], preferred_element_type=jnp.float32) + br2_ref[...]
    gid = lax.broadcasted_iota(jnp.int32, (1, NG), 1)
    oh = (batch_ref[...] == gid).astype(jnp.float32)
    sums = lax.dot_general(
        oh, nf, (((0,), (0,)), ((), ())), preferred_element_type=jnp.float32
    )
    counts = lax.dot_general(
        oh,
        jnp.ones((N, 1), jnp.float32),
        (((0,), (0,)), ((), ())),
        preferred_element_type=jnp.float32,
    )
    g = sums / jnp.maximum(counts, 1.0)
    o_ref[...] = (
        jnp.dot(g, wp_ref[...], preferred_element_type=jnp.float32) + bp_ref[...]
    )


def _tc_readout(h, batch2d, W_r1, b_r1, W_r2, b_r2, W_p, b_p):
    return pl.pallas_call(
        _readout_body,
        out_shape=jax.ShapeDtypeStruct((NG, 1), jnp.float32),
    )(h, batch2d, W_r1, b_r1.reshape(1, D), W_r2, b_r2.reshape(1, D),
      W_p, b_p.reshape(1, 1))


def kernel(x, edge_index, edge_attr, batch,
           W_proj, b_proj, W_e1, b_e1, W_e2, b_e2, W_root, b_conv,
           W_gru_ih, b_gru_ih, W_gru_hh, b_gru_hh,
           W_r1, b_r1, W_r2, b_r2, W_p, b_p):
    src3 = edge_index[0].reshape(NW, NCH, CW)
    dst3 = edge_index[1].reshape(NW, NCH, CW)
    batch2d = batch.reshape(N, 1)
    zeros_nd = jnp.zeros((N, D), jnp.float32)

    eap = edge_attr.reshape(E // 16, 16 * 16)
    eye16 = jnp.eye(16, dtype=jnp.float32)
    BD1 = jnp.kron(eye16, W_e1)
    b1t = jnp.tile(b_e1, 16).reshape(1, 256)
    BD2 = jnp.concatenate(
        [jnp.kron(eye16, W_e2[:, D * i:D * (i + 1)]) for i in range(D)], axis=1
    )
    b2t = jnp.stack([jnp.tile(b_e2[D * i:D * (i + 1)], 16) for i in range(D)])
    onehot8 = jnp.eye(D, dtype=jnp.float32)
    G = jnp.concatenate(
        [jnp.kron(eye16, onehot8[:, i:i + 1] * jnp.ones((1, D), jnp.float32))
         for i in range(D)], axis=1
    )

    gru_w = (
        W_gru_ih[:, 0:D], W_gru_ih[:, D:2 * D], W_gru_ih[:, 2 * D:3 * D],
        b_gru_ih[0:D].reshape(1, D), b_gru_ih[D:2 * D].reshape(1, D),
        b_gru_ih[2 * D:3 * D].reshape(1, D),
        W_gru_hh[:, 0:D], W_gru_hh[:, D:2 * D], W_gru_hh[:, 2 * D:3 * D],
        b_gru_hh[0:D].reshape(1, D), b_gru_hh[D:2 * D].reshape(1, D),
        b_gru_hh[2 * D:3 * D].reshape(1, D),
    )

    h = _tc_project(x, W_proj, b_proj)
    hidden = h
    for _ in range(STEPS):
        h_src = _sc_gather(h, src3)
        msg_p = _tc_message(eap, h_src.reshape(E // 16, 128),
                            BD1, b1t, BD2, b2t, G)
        agg2 = _sc_scatter_add(msg_p.reshape(E, D), dst3, zeros_nd)
        hidden = _tc_gru(agg2, h, hidden, W_root, b_conv, gru_w)
        h = hidden
    return _tc_readout(h, batch2d, W_r1, b_r1, W_r2, b_r2, W_p, b_p)


# speedup vs baseline: 4.2014x; 1.1140x over previous
import functools

import jax
import jax.numpy as jnp
from jax import lax
from jax.experimental import pallas as pl
from jax.experimental.pallas import tpu as pltpu
from jax.experimental.pallas import tpu_sc as plsc

N = 10000
E = 320000
D = 8
NG = 64
STEPS = 3

NC = 2
NS = 16
NW = NC * NS
EPW = E // NW
CW = 125
SUB = 8
SCW = CW * SUB
NSC = EPW // SCW
NCH = EPW // CW
ROWS_PER_TILE = N // NS

_SC_PARAMS = pltpu.CompilerParams(use_tc_tiling_on_sc=False)


@functools.cache
def _sc_mesh():
    return plsc.VectorSubcoreMesh(
        core_axis_name="c", subcore_axis_name="s", num_cores=NC, num_subcores=NS
    )


def _gather_body(h_hbm, src_hbm, out_hbm, idx_v, rows_v, gsem, ssem):
    wid = lax.axis_index("s") * NC + lax.axis_index("c")
    base = wid * EPW
    pltpu.sync_copy(src_hbm.at[wid], idx_v)

    def gathers(m, half):
        descs = []
        for b in range(SUB):
            descs.append(pltpu.async_copy(
                h_hbm.at[idx_v.at[m * SUB + b]],
                rows_v.at[half, pl.ds(b * CW, CW)],
                gsem,
            ))
        return descs

    def store_desc(m, half):
        return pltpu.make_async_copy(
            rows_v.at[half], out_hbm.at[pl.ds(base + m * SCW, SCW)], ssem
        )

    def body(m, carry):
        half = lax.rem(m, 2)

        @pl.when(m >= 2)
        def _():
            store_desc(m - 2, half).wait()

        descs = gathers(m, half)
        for dsc in descs:
            dsc.wait()
        pltpu.async_copy(
            rows_v.at[half], out_hbm.at[pl.ds(base + m * SCW, SCW)], ssem
        )
        return carry

    lax.fori_loop(0, NSC, body, 0)
    store_desc(NSC - 2, lax.rem(NSC - 2, 2)).wait()
    store_desc(NSC - 1, lax.rem(NSC - 1, 2)).wait()


def _sc_gather(h, src3):
    kern = pl.kernel(
        _gather_body,
        out_type=jax.ShapeDtypeStruct((E, D), jnp.float32),
        mesh=_sc_mesh(),
        scratch_types=[
            pltpu.VMEM((NCH, CW), jnp.int32),
            pltpu.VMEM((2, SCW, D), jnp.float32),
            pltpu.SemaphoreType.DMA,
            pltpu.SemaphoreType.DMA,
        ],
        compiler_params=_SC_PARAMS,
    )
    return kern(h, src3)


def _scatter_body(msg_hbm, dst_hbm, zeros_hbm, out_hbm, didx_v, mrows_v, agg_sh,
                  lsem, asem):
    cid = lax.axis_index("c")
    sid = lax.axis_index("s")
    wid = sid * NC + cid
    base = wid * EPW

    pltpu.sync_copy(
        zeros_hbm.at[pl.ds(sid * ROWS_PER_TILE, ROWS_PER_TILE)],
        agg_sh.at[pl.ds(sid * ROWS_PER_TILE, ROWS_PER_TILE)],
    )
    pltpu.sync_copy(dst_hbm.at[wid], didx_v)
    plsc.subcore_barrier()

    def load_desc(m, half):
        return pltpu.make_async_copy(
            msg_hbm.at[pl.ds(base + m * SCW, SCW)], mrows_v.at[half], lsem
        )

    def scat_desc(m, half, b):
        return pltpu.make_async_copy(
            mrows_v.at[half, pl.ds(b * CW, CW)],
            agg_sh.at[didx_v.at[m * SUB + b]],
            asem,
        )

    def body(m, carry):
        half = lax.rem(m, 2)

        @pl.when(m >= 2)
        def _():
            for b in range(SUB):
                scat_desc(m - 2, half, b).wait()

        load_desc(m, half).start()
        load_desc(m, half).wait()
        for b in range(SUB):
            pltpu.async_copy(
                mrows_v.at[half, pl.ds(b * CW, CW)],
                agg_sh.at[didx_v.at[m * SUB + b]],
                asem,
                add=True,
            )
        return carry

    lax.fori_loop(0, NSC, body, 0)
    for m in (NSC - 2, NSC - 1):
        for b in range(SUB):
            scat_desc(m, m % 2, b).wait()
    plsc.subcore_barrier()

    pltpu.sync_copy(
        agg_sh.at[pl.ds(sid * ROWS_PER_TILE, ROWS_PER_TILE)],
        out_hbm.at[cid, pl.ds(sid * ROWS_PER_TILE, ROWS_PER_TILE)],
    )


def _sc_scatter_add(msg, dst3, zeros_nd):
    kern = pl.kernel(
        _scatter_body,
        out_type=jax.ShapeDtypeStruct((NC, N, D), jnp.float32),
        mesh=_sc_mesh(),
        scratch_types=[
            pltpu.VMEM((NCH, CW), jnp.int32),
            pltpu.VMEM((2, SCW, D), jnp.float32),
            pltpu.VMEM_SHARED((N, D), jnp.float32),
            pltpu.SemaphoreType.DMA,
            pltpu.SemaphoreType.DMA,
        ],
        compiler_params=_SC_PARAMS,
    )
    return kern(msg, dst3, zeros_nd)


def _proj_body(x_ref, w_ref, b_ref, o_ref):
    o_ref[...] = jnp.maximum(
        jnp.dot(x_ref[...], w_ref[...], preferred_element_type=jnp.float32)
        + b_ref[...],
        0.0,
    )


def _tc_project(x, W_proj, b_proj):
    return pl.pallas_call(
        _proj_body,
        out_shape=jax.ShapeDtypeStruct((N, D), jnp.float32),
    )(x, W_proj, b_proj.reshape(1, D))


MSG_BM = 16000


def _msg_body(eap_ref, hsp_ref, bd1_ref, b1t_ref, bd2_ref, b2t_ref, g_ref, o_ref):
    ehp = jnp.maximum(
        jnp.dot(eap_ref[...], bd1_ref[...], preferred_element_type=jnp.float32)
        + b1t_ref[...],
        0.0,
    )
    ew2 = jnp.dot(ehp, bd2_ref[...], preferred_element_type=jnp.float32)
    hs2 = jnp.dot(hsp_ref[...], g_ref[...], preferred_element_type=jnp.float32)
    b2t = b2t_ref[...]
    acc = hs2[:, 0:128] * (ew2[:, 0:128] + b2t[0:1, :])
    for i in range(1, D):
        acc += hs2[:, 128 * i:128 * (i + 1)] * (
            ew2[:, 128 * i:128 * (i + 1)] + b2t[i:i + 1, :]
        )
    o_ref[...] = acc


def _tc_message(eap, h_src_p, BD1, b1t, BD2, b2t, G):
    grid = E // MSG_BM
    pb = MSG_BM // 16
    return pl.pallas_call(
        _msg_body,
        grid=(grid,),
        in_specs=[
            pl.BlockSpec((pb, 256), lambda i: (i, 0)),
            pl.BlockSpec((pb, 128), lambda i: (i, 0)),
            pl.BlockSpec((256, 256), lambda i: (0, 0)),
            pl.BlockSpec((1, 256), lambda i: (0, 0)),
            pl.BlockSpec((256, D * 128), lambda i: (0, 0)),
            pl.BlockSpec((D, 128), lambda i: (0, 0)),
            pl.BlockSpec((128, D * 128), lambda i: (0, 0)),
        ],
        out_specs=pl.BlockSpec((pb, 128), lambda i: (i, 0)),
        out_shape=jax.ShapeDtypeStruct((E // 16, 128), jnp.float32),
    )(eap, h_src_p, BD1, b1t, BD2, b2t, G)


NP = N // 16


def _gru_body(
    agg2_ref, h_ref, hid_ref, wroot_ref, bconv_ref,
    wir_ref, wiz_ref, win_ref, bir_ref, biz_ref, bin_ref,
    whr_ref, whz_ref, whn_ref, bhr_ref, bhz_ref, bhn_ref,
    o_ref,
):
    agg = agg2_ref[0] + agg2_ref[1]
    h = h_ref[...]
    hidden = hid_ref[...]
    m = jnp.maximum(
        agg
        + jnp.dot(h, wroot_ref[...], preferred_element_type=jnp.float32)
        + bconv_ref[...],
        0.0,
    )
    i_r = jnp.dot(m, wir_ref[...], preferred_element_type=jnp.float32) + bir_ref[...]
    i_z = jnp.dot(m, wiz_ref[...], preferred_element_type=jnp.float32) + biz_ref[...]
    i_n = jnp.dot(m, win_ref[...], preferred_element_type=jnp.float32) + bin_ref[...]
    h_r = jnp.dot(hidden, whr_ref[...], preferred_element_type=jnp.float32) + bhr_ref[...]
    h_z = jnp.dot(hidden, whz_ref[...], preferred_element_type=jnp.float32) + bhz_ref[...]
    h_n = jnp.dot(hidden, whn_ref[...], preferred_element_type=jnp.float32) + bhn_ref[...]
    r = jax.nn.sigmoid(i_r + h_r)
    z = jax.nn.sigmoid(i_z + h_z)
    n = jnp.tanh(i_n + r * h_n)
    o_ref[...] = (1.0 - z) * n + z * hidden


def _tc_gru(agg2p, hp, hiddenp, kron_w):
    return pl.pallas_call(
        _gru_body,
        out_shape=jax.ShapeDtypeStruct((NP, 128), jnp.float32),
    )(agg2p, hp, hiddenp, *kron_w)


def _readout_body(
    hp_ref, batchp_ref, kr1_ref, br1_ref, kr2_ref, br2_ref, wp_ref, bp_ref, o_ref
):
    hp = hp_ref[...]
    nfp = jnp.maximum(
        jnp.dot(hp, kr1_ref[...], preferred_element_type=jnp.float32)
        + br1_ref[...],
        0.0,
    )
    nfp = jnp.dot(nfp, kr2_ref[...], preferred_element_type=jnp.float32) + br2_ref[...]
    batchp = batchp_ref[...]
    gid = lax.broadcasted_iota(jnp.int32, (1, NG), 1)
    sums = jnp.zeros((NG, D), jnp.float32)
    oh_sum = jnp.zeros((NP, NG), jnp.float32)
    for k in range(16):
        ohk = (batchp[:, k:k + 1] == gid).astype(jnp.float32)
        oh_sum = oh_sum + ohk
        sk = lax.dot_general(
            ohk, nfp, (((0,), (0,)), ((), ())), preferred_element_type=jnp.float32
        )
        sums = sums + sk[:, D * k:D * (k + 1)]
    counts = lax.dot_general(
        oh_sum, jnp.ones((NP, 1), jnp.float32), (((0,), (0,)), ((), ())),
        preferred_element_type=jnp.float32,
    )
    g = sums / jnp.maximum(counts, 1.0)
    o_ref[...] = (
        jnp.dot(g, wp_ref[...], preferred_element_type=jnp.float32) + bp_ref[...]
    )


def _tc_readout(hp, batchp, W_r1, b_r1, W_r2, b_r2, W_p, b_p):
    eye16 = jnp.eye(16, dtype=jnp.float32)
    return pl.pallas_call(
        _readout_body,
        out_shape=jax.ShapeDtypeStruct((NG, 1), jnp.float32),
    )(hp, batchp, jnp.kron(eye16, W_r1), jnp.tile(b_r1, 16).reshape(1, 128),
      jnp.kron(eye16, W_r2), jnp.tile(b_r2, 16).reshape(1, 128),
      W_p, b_p.reshape(1, 1))


def kernel(x, edge_index, edge_attr, batch,
           W_proj, b_proj, W_e1, b_e1, W_e2, b_e2, W_root, b_conv,
           W_gru_ih, b_gru_ih, W_gru_hh, b_gru_hh,
           W_r1, b_r1, W_r2, b_r2, W_p, b_p):
    src3 = edge_index[0].reshape(NW, NCH, CW)
    dst3 = edge_index[1].reshape(NW, NCH, CW)
    batch2d = batch.reshape(N, 1)
    zeros_nd = jnp.zeros((N, D), jnp.float32)

    eap = edge_attr.reshape(E // 16, 16 * 16)
    eye16 = jnp.eye(16, dtype=jnp.float32)
    BD1 = jnp.kron(eye16, W_e1)
    b1t = jnp.tile(b_e1, 16).reshape(1, 256)
    BD2 = jnp.concatenate(
        [jnp.kron(eye16, W_e2[:, D * i:D * (i + 1)]) for i in range(D)], axis=1
    )
    b2t = jnp.stack([jnp.tile(b_e2[D * i:D * (i + 1)], 16) for i in range(D)])
    onehot8 = jnp.eye(D, dtype=jnp.float32)
    G = jnp.concatenate(
        [jnp.kron(eye16, onehot8[:, i:i + 1] * jnp.ones((1, D), jnp.float32))
         for i in range(D)], axis=1
    )

    def kt(w):
        return jnp.kron(eye16, w)

    def bt(b):
        return jnp.tile(b, 16).reshape(1, 128)

    kron_w = (
        kt(W_root), bt(b_conv),
        kt(W_gru_ih[:, 0:D]), kt(W_gru_ih[:, D:2 * D]), kt(W_gru_ih[:, 2 * D:]),
        bt(b_gru_ih[0:D]), bt(b_gru_ih[D:2 * D]), bt(b_gru_ih[2 * D:]),
        kt(W_gru_hh[:, 0:D]), kt(W_gru_hh[:, D:2 * D]), kt(W_gru_hh[:, 2 * D:]),
        bt(b_gru_hh[0:D]), bt(b_gru_hh[D:2 * D]), bt(b_gru_hh[2 * D:]),
    )

    hp = _tc_project(x, W_proj, b_proj).reshape(NP, 128)
    hiddenp = hp
    for _ in range(STEPS):
        h_src = _sc_gather(hp.reshape(N, D), src3)
        msg_p = _tc_message(eap, h_src.reshape(E // 16, 128),
                            BD1, b1t, BD2, b2t, G)
        agg2 = _sc_scatter_add(msg_p.reshape(E, D), dst3, zeros_nd)
        hiddenp = _tc_gru(agg2.reshape(NC, NP, 128), hp, hiddenp, kron_w)
        hp = hiddenp
    return _tc_readout(hp, batch.reshape(NP, 16), W_r1, b_r1, W_r2, b_r2, W_p, b_p)


# speedup vs baseline: 4.3861x; 1.0440x over previous
import functools

import jax
import jax.numpy as jnp
from jax import lax
from jax.experimental import pallas as pl
from jax.experimental.pallas import tpu as pltpu
from jax.experimental.pallas import tpu_sc as plsc

N = 10000
E = 320000
D = 8
NG = 64
STEPS = 3

NC = 2
NS = 16
NW = NC * NS
EPW = E // NW
CW = 125
SUB = 8
SCW = CW * SUB
NSC = EPW // SCW
NCH = EPW // CW
ROWS_PER_TILE = N // NS

_SC_PARAMS = pltpu.CompilerParams(use_tc_tiling_on_sc=False)


@functools.cache
def _sc_mesh():
    return plsc.VectorSubcoreMesh(
        core_axis_name="c", subcore_axis_name="s", num_cores=NC, num_subcores=NS
    )


def _gather_body(h_hbm, src_hbm, out_hbm, idx_v, rows_v, gsem, ssem):
    wid = lax.axis_index("s") * NC + lax.axis_index("c")
    base = wid * EPW
    pltpu.sync_copy(src_hbm.at[wid], idx_v)

    def gathers(m, half):
        descs = []
        for b in range(SUB):
            descs.append(pltpu.async_copy(
                h_hbm.at[idx_v.at[m * SUB + b]],
                rows_v.at[half, pl.ds(b * CW, CW)],
                gsem,
            ))
        return descs

    def store_desc(m, half):
        return pltpu.make_async_copy(
            rows_v.at[half], out_hbm.at[pl.ds(base + m * SCW, SCW)], ssem
        )

    def body(m, carry):
        half = lax.rem(m, 2)

        @pl.when(m >= 2)
        def _():
            store_desc(m - 2, half).wait()

        descs = gathers(m, half)
        for dsc in descs:
            dsc.wait()
        pltpu.async_copy(
            rows_v.at[half], out_hbm.at[pl.ds(base + m * SCW, SCW)], ssem
        )
        return carry

    lax.fori_loop(0, NSC, body, 0)
    store_desc(NSC - 2, lax.rem(NSC - 2, 2)).wait()
    store_desc(NSC - 1, lax.rem(NSC - 1, 2)).wait()


def _sc_gather(h, src3):
    kern = pl.kernel(
        _gather_body,
        out_type=jax.ShapeDtypeStruct((E, D), jnp.float32),
        mesh=_sc_mesh(),
        scratch_types=[
            pltpu.VMEM((NCH, CW), jnp.int32),
            pltpu.VMEM((2, SCW, D), jnp.float32),
            pltpu.SemaphoreType.DMA,
            pltpu.SemaphoreType.DMA,
        ],
        compiler_params=_SC_PARAMS,
    )
    return kern(h, src3)


def _scatter_body(msg_hbm, dst_hbm, zeros_hbm, out_hbm, didx_v, mrows_v, agg_sh,
                  lsem, asem):
    cid = lax.axis_index("c")
    sid = lax.axis_index("s")
    wid = sid * NC + cid
    base = wid * EPW

    pltpu.sync_copy(
        zeros_hbm.at[pl.ds(sid * ROWS_PER_TILE, ROWS_PER_TILE)],
        agg_sh.at[pl.ds(sid * ROWS_PER_TILE, ROWS_PER_TILE)],
    )
    pltpu.sync_copy(dst_hbm.at[wid], didx_v)
    plsc.subcore_barrier()

    def load_desc(m, half):
        return pltpu.make_async_copy(
            msg_hbm.at[pl.ds(base + m * SCW, SCW)], mrows_v.at[half], lsem
        )

    def scat_desc(m, half, b):
        return pltpu.make_async_copy(
            mrows_v.at[half, pl.ds(b * CW, CW)],
            agg_sh.at[didx_v.at[m * SUB + b]],
            asem,
        )

    def body(m, carry):
        half = lax.rem(m, 2)

        @pl.when(m >= 2)
        def _():
            for b in range(SUB):
                scat_desc(m - 2, half, b).wait()

        load_desc(m, half).start()
        load_desc(m, half).wait()
        for b in range(SUB):
            pltpu.async_copy(
                mrows_v.at[half, pl.ds(b * CW, CW)],
                agg_sh.at[didx_v.at[m * SUB + b]],
                asem,
                add=True,
            )
        return carry

    lax.fori_loop(0, NSC, body, 0)
    for m in (NSC - 2, NSC - 1):
        for b in range(SUB):
            scat_desc(m, m % 2, b).wait()
    plsc.subcore_barrier()

    pltpu.sync_copy(
        agg_sh.at[pl.ds(sid * ROWS_PER_TILE, ROWS_PER_TILE)],
        out_hbm.at[cid, pl.ds(sid * ROWS_PER_TILE, ROWS_PER_TILE)],
    )


def _sc_scatter_add(msg, dst3, zeros_nd):
    kern = pl.kernel(
        _scatter_body,
        out_type=jax.ShapeDtypeStruct((NC, N, D), jnp.float32),
        mesh=_sc_mesh(),
        scratch_types=[
            pltpu.VMEM((NCH, CW), jnp.int32),
            pltpu.VMEM((2, SCW, D), jnp.float32),
            pltpu.VMEM_SHARED((N, D), jnp.float32),
            pltpu.SemaphoreType.DMA,
            pltpu.SemaphoreType.DMA,
        ],
        compiler_params=_SC_PARAMS,
    )
    return kern(msg, dst3, zeros_nd)


def _proj_body(x_ref, w_ref, b_ref, o_ref):
    o_ref[...] = jnp.maximum(
        jnp.dot(x_ref[...], w_ref[...], preferred_element_type=jnp.float32)
        + b_ref[...],
        0.0,
    )


def _tc_project(x, W_proj, b_proj):
    return pl.pallas_call(
        _proj_body,
        out_shape=jax.ShapeDtypeStruct((N, D), jnp.float32),
    )(x, W_proj, b_proj.reshape(1, D))


MSG_BM = 16000


EHP_BM = 6400


def _ehp_body(eat_ref, bd1_ref, b1t_ref, o_ref):
    ea = eat_ref[...].T
    ea3 = ea.reshape(EHP_BM // 16, 16, 16)
    eap = jnp.concatenate([ea3[:, k, :] for k in range(16)], axis=1)
    o_ref[...] = jnp.maximum(
        jnp.dot(eap, bd1_ref[...], preferred_element_type=jnp.float32)
        + b1t_ref[...],
        0.0,
    )


def _tc_ehp(eaT, BD1, b1t):
    grid = E // EHP_BM
    return pl.pallas_call(
        _ehp_body,
        grid=(grid,),
        in_specs=[
            pl.BlockSpec((16, EHP_BM), lambda i: (0, i)),
            pl.BlockSpec((256, 256), lambda i: (0, 0)),
            pl.BlockSpec((1, 256), lambda i: (0, 0)),
        ],
        out_specs=pl.BlockSpec((EHP_BM // 16, 256), lambda i: (i, 0)),
        out_shape=jax.ShapeDtypeStruct((E // 16, 256), jnp.float32),
    )(eaT, BD1, b1t)


def _msg_body(ehp_ref, hsp_ref, bd2_ref, b2t_ref, g_ref, o_ref):
    ehp = ehp_ref[...]
    ew2 = jnp.dot(ehp, bd2_ref[...], preferred_element_type=jnp.float32)
    hs2 = jnp.dot(hsp_ref[...], g_ref[...], preferred_element_type=jnp.float32)
    b2t = b2t_ref[...]
    acc = hs2[:, 0:128] * (ew2[:, 0:128] + b2t[0:1, :])
    for i in range(1, D):
        acc += hs2[:, 128 * i:128 * (i + 1)] * (
            ew2[:, 128 * i:128 * (i + 1)] + b2t[i:i + 1, :]
        )
    o_ref[...] = acc


def _tc_message(ehp, h_src_p, BD2, b2t, G):
    grid = E // MSG_BM
    pb = MSG_BM // 16
    return pl.pallas_call(
        _msg_body,
        grid=(grid,),
        in_specs=[
            pl.BlockSpec((pb, 256), lambda i: (i, 0)),
            pl.BlockSpec((pb, 128), lambda i: (i, 0)),
            pl.BlockSpec((256, D * 128), lambda i: (0, 0)),
            pl.BlockSpec((D, 128), lambda i: (0, 0)),
            pl.BlockSpec((128, D * 128), lambda i: (0, 0)),
        ],
        out_specs=pl.BlockSpec((pb, 128), lambda i: (i, 0)),
        out_shape=jax.ShapeDtypeStruct((E // 16, 128), jnp.float32),
    )(ehp, h_src_p, BD2, b2t, G)


NP = N // 16


def _gru_body(
    agg2_ref, h_ref, hid_ref, wroot_ref, bconv_ref,
    wir_ref, wiz_ref, win_ref, bir_ref, biz_ref, bin_ref,
    whr_ref, whz_ref, whn_ref, bhr_ref, bhz_ref, bhn_ref,
    o_ref,
):
    agg = agg2_ref[0] + agg2_ref[1]
    h = h_ref[...]
    hidden = hid_ref[...]
    m = jnp.maximum(
        agg
        + jnp.dot(h, wroot_ref[...], preferred_element_type=jnp.float32)
        + bconv_ref[...],
        0.0,
    )
    i_r = jnp.dot(m, wir_ref[...], preferred_element_type=jnp.float32) + bir_ref[...]
    i_z = jnp.dot(m, wiz_ref[...], preferred_element_type=jnp.float32) + biz_ref[...]
    i_n = jnp.dot(m, win_ref[...], preferred_element_type=jnp.float32) + bin_ref[...]
    h_r = jnp.dot(hidden, whr_ref[...], preferred_element_type=jnp.float32) + bhr_ref[...]
    h_z = jnp.dot(hidden, whz_ref[...], preferred_element_type=jnp.float32) + bhz_ref[...]
    h_n = jnp.dot(hidden, whn_ref[...], preferred_element_type=jnp.float32) + bhn_ref[...]
    r = jax.nn.sigmoid(i_r + h_r)
    z = jax.nn.sigmoid(i_z + h_z)
    n = jnp.tanh(i_n + r * h_n)
    o_ref[...] = (1.0 - z) * n + z * hidden


def _tc_gru(agg2p, hp, hiddenp, kron_w):
    return pl.pallas_call(
        _gru_body,
        out_shape=jax.ShapeDtypeStruct((NP, 128), jnp.float32),
    )(agg2p, hp, hiddenp, *kron_w)


def _readout_body(
    hp_ref, batchp_ref, kr1_ref, br1_ref, kr2_ref, br2_ref, wp_ref, bp_ref, o_ref
):
    hp = hp_ref[...]
    nfp = jnp.maximum(
        jnp.dot(hp, kr1_ref[...], preferred_element_type=jnp.float32)
        + br1_ref[...],
        0.0,
    )
    nfp = jnp.dot(nfp, kr2_ref[...], preferred_element_type=jnp.float32) + br2_ref[...]
    batchp = batchp_ref[...]
    gid = lax.broadcasted_iota(jnp.int32, (1, NG), 1)
    sums = jnp.zeros((NG, D), jnp.float32)
    oh_sum = jnp.zeros((NP, NG), jnp.float32)
    for k in range(16):
        ohk = (batchp[:, k:k + 1] == gid).astype(jnp.float32)
        oh_sum = oh_sum + ohk
        sk = lax.dot_general(
            ohk, nfp, (((0,), (0,)), ((), ())), preferred_element_type=jnp.float32
        )
        sums = sums + sk[:, D * k:D * (k + 1)]
    counts = lax.dot_general(
        oh_sum, jnp.ones((NP, 1), jnp.float32), (((0,), (0,)), ((), ())),
        preferred_element_type=jnp.float32,
    )
    g = sums / jnp.maximum(counts, 1.0)
    o_ref[...] = (
        jnp.dot(g, wp_ref[...], preferred_element_type=jnp.float32) + bp_ref[...]
    )


def _tc_readout(hp, batchp, W_r1, b_r1, W_r2, b_r2, W_p, b_p):
    eye16 = jnp.eye(16, dtype=jnp.float32)
    return pl.pallas_call(
        _readout_body,
        out_shape=jax.ShapeDtypeStruct((NG, 1), jnp.float32),
    )(hp, batchp, jnp.kron(eye16, W_r1), jnp.tile(b_r1, 16).reshape(1, 128),
      jnp.kron(eye16, W_r2), jnp.tile(b_r2, 16).reshape(1, 128),
      W_p, b_p.reshape(1, 1))


def kernel(x, edge_index, edge_attr, batch,
           W_proj, b_proj, W_e1, b_e1, W_e2, b_e2, W_root, b_conv,
           W_gru_ih, b_gru_ih, W_gru_hh, b_gru_hh,
           W_r1, b_r1, W_r2, b_r2, W_p, b_p):
    src3 = edge_index[0].reshape(NW, NCH, CW)
    dst3 = edge_index[1].reshape(NW, NCH, CW)
    batch2d = batch.reshape(N, 1)
    zeros_nd = jnp.zeros((N, D), jnp.float32)

    eye16 = jnp.eye(16, dtype=jnp.float32)
    BD1 = jnp.kron(eye16, W_e1)
    b1t = jnp.tile(b_e1, 16).reshape(1, 256)
    BD2 = jnp.concatenate(
        [jnp.kron(eye16, W_e2[:, D * i:D * (i + 1)]) for i in range(D)], axis=1
    )
    b2t = jnp.stack([jnp.tile(b_e2[D * i:D * (i + 1)], 16) for i in range(D)])
    onehot8 = jnp.eye(D, dtype=jnp.float32)
    G = jnp.concatenate(
        [jnp.kron(eye16, onehot8[:, i:i + 1] * jnp.ones((1, D), jnp.float32))
         for i in range(D)], axis=1
    )

    def kt(w):
        return jnp.kron(eye16, w)

    def bt(b):
        return jnp.tile(b, 16).reshape(1, 128)

    kron_w = (
        kt(W_root), bt(b_conv),
        kt(W_gru_ih[:, 0:D]), kt(W_gru_ih[:, D:2 * D]), kt(W_gru_ih[:, 2 * D:]),
        bt(b_gru_ih[0:D]), bt(b_gru_ih[D:2 * D]), bt(b_gru_ih[2 * D:]),
        kt(W_gru_hh[:, 0:D]), kt(W_gru_hh[:, D:2 * D]), kt(W_gru_hh[:, 2 * D:]),
        bt(b_gru_hh[0:D]), bt(b_gru_hh[D:2 * D]), bt(b_gru_hh[2 * D:]),
    )

    ehp = _tc_ehp(edge_attr.T, BD1, b1t)
    hp = _tc_project(x, W_proj, b_proj).reshape(NP, 128)
    hiddenp = hp
    for _ in range(STEPS):
        h_src = _sc_gather(hp.reshape(N, D), src3)
        msg_p = _tc_message(ehp, h_src.reshape(E // 16, 128),
                            BD2, b2t, G)
        agg2 = _sc_scatter_add(msg_p.reshape(E, D), dst3, zeros_nd)
        hiddenp = _tc_gru(agg2.reshape(NC, NP, 128), hp, hiddenp, kron_w)
        hp = hiddenp
    return _tc_readout(hp, batch.reshape(NP, 16), W_r1, b_r1, W_r2, b_r2, W_p, b_p)


# speedup vs baseline: 4.4307x; 1.0102x over previous
import functools

import jax
import jax.numpy as jnp
from jax import lax
from jax.experimental import pallas as pl
from jax.experimental.pallas import tpu as pltpu
from jax.experimental.pallas import tpu_sc as plsc

N = 10000
E = 320000
D = 8
NG = 64
STEPS = 3

NC = 2
NS = 16
NW = NC * NS
EPW = E // NW
CW = 125
SUB = 8
SCW = CW * SUB
NSC = EPW // SCW
NCH = EPW // CW
ROWS_PER_TILE = N // NS

_SC_PARAMS = pltpu.CompilerParams(use_tc_tiling_on_sc=False)


@functools.cache
def _sc_mesh():
    return plsc.VectorSubcoreMesh(
        core_axis_name="c", subcore_axis_name="s", num_cores=NC, num_subcores=NS
    )


def _gather_body(h_hbm, src_hbm, out_hbm, idx_v, rows_v, gsem, ssem):
    wid = lax.axis_index("s") * NC + lax.axis_index("c")
    base = wid * EPW
    pltpu.sync_copy(src_hbm.at[wid], idx_v)

    def gathers(m, half):
        descs = []
        for b in range(SUB):
            descs.append(pltpu.async_copy(
                h_hbm.at[idx_v.at[m * SUB + b]],
                rows_v.at[half, pl.ds(b * CW, CW)],
                gsem,
            ))
        return descs

    def store_desc(m, half):
        return pltpu.make_async_copy(
            rows_v.at[half], out_hbm.at[pl.ds(base + m * SCW, SCW)], ssem
        )

    def body(m, carry):
        half = lax.rem(m, 2)

        @pl.when(m >= 2)
        def _():
            store_desc(m - 2, half).wait()

        descs = gathers(m, half)
        for dsc in descs:
            dsc.wait()
        pltpu.async_copy(
            rows_v.at[half], out_hbm.at[pl.ds(base + m * SCW, SCW)], ssem
        )
        return carry

    lax.fori_loop(0, NSC, body, 0)
    store_desc(NSC - 2, lax.rem(NSC - 2, 2)).wait()
    store_desc(NSC - 1, lax.rem(NSC - 1, 2)).wait()


def _sc_gather(h, src3):
    kern = pl.kernel(
        _gather_body,
        out_type=jax.ShapeDtypeStruct((E, D), jnp.float32),
        mesh=_sc_mesh(),
        scratch_types=[
            pltpu.VMEM((NCH, CW), jnp.int32),
            pltpu.VMEM((2, SCW, D), jnp.float32),
            pltpu.SemaphoreType.DMA,
            pltpu.SemaphoreType.DMA,
        ],
        compiler_params=_SC_PARAMS,
    )
    return kern(h, src3)


def _scatter_body(msg_hbm, dst_hbm, zeros_hbm, out_hbm, didx_v, mrows_v, agg_sh,
                  lsem, asem):
    cid = lax.axis_index("c")
    sid = lax.axis_index("s")
    wid = sid * NC + cid
    base = wid * EPW

    pltpu.sync_copy(
        zeros_hbm.at[pl.ds(sid * ROWS_PER_TILE, ROWS_PER_TILE)],
        agg_sh.at[pl.ds(sid * ROWS_PER_TILE, ROWS_PER_TILE)],
    )
    pltpu.sync_copy(dst_hbm.at[wid], didx_v)
    plsc.subcore_barrier()

    def load_desc(m, half):
        return pltpu.make_async_copy(
            msg_hbm.at[pl.ds(base + m * SCW, SCW)], mrows_v.at[half], lsem
        )

    def scat_desc(m, half, b):
        return pltpu.make_async_copy(
            mrows_v.at[half, pl.ds(b * CW, CW)],
            agg_sh.at[didx_v.at[m * SUB + b]],
            asem,
        )

    def body(m, carry):
        half = lax.rem(m, 2)

        @pl.when(m >= 2)
        def _():
            for b in range(SUB):
                scat_desc(m - 2, half, b).wait()

        load_desc(m, half).start()
        load_desc(m, half).wait()
        for b in range(SUB):
            pltpu.async_copy(
                mrows_v.at[half, pl.ds(b * CW, CW)],
                agg_sh.at[didx_v.at[m * SUB + b]],
                asem,
                add=True,
            )
        return carry

    lax.fori_loop(0, NSC, body, 0)
    for m in (NSC - 2, NSC - 1):
        for b in range(SUB):
            scat_desc(m, m % 2, b).wait()
    plsc.subcore_barrier()

    pltpu.sync_copy(
        agg_sh.at[pl.ds(sid * ROWS_PER_TILE, ROWS_PER_TILE)],
        out_hbm.at[cid, pl.ds(sid * ROWS_PER_TILE, ROWS_PER_TILE)],
    )


def _sc_scatter_add(msg, dst3, zeros_nd):
    kern = pl.kernel(
        _scatter_body,
        out_type=jax.ShapeDtypeStruct((NC, N, D), jnp.float32),
        mesh=_sc_mesh(),
        scratch_types=[
            pltpu.VMEM((NCH, CW), jnp.int32),
            pltpu.VMEM((2, SCW, D), jnp.float32),
            pltpu.VMEM_SHARED((N, D), jnp.float32),
            pltpu.SemaphoreType.DMA,
            pltpu.SemaphoreType.DMA,
        ],
        compiler_params=_SC_PARAMS,
    )
    return kern(msg, dst3, zeros_nd)


def _proj_body(x_ref, w_ref, b_ref, o_ref):
    o_ref[...] = jnp.maximum(
        jnp.dot(x_ref[...], w_ref[...], preferred_element_type=jnp.float32)
        + b_ref[...],
        0.0,
    )


def _tc_project(x, W_proj, b_proj):
    return pl.pallas_call(
        _proj_body,
        out_shape=jax.ShapeDtypeStruct((N, D), jnp.float32),
    )(x, W_proj, b_proj.reshape(1, D))


MSG_BM = 16000


EHP_BM = 6400


def _ehp_body(eat_ref, we1_ref, be1_ref, o_ref, s3_ref):
    eh = jnp.maximum(
        lax.dot_general(
            eat_ref[...], we1_ref[...], (((0,), (0,)), ((), ())),
            preferred_element_type=jnp.float32,
        )
        + be1_ref[...],
        0.0,
    )
    s3_ref[...] = eh.reshape(EHP_BM // 16, 16, 16)
    o_ref[...] = s3_ref[...].reshape(EHP_BM // 16, 256)


def _tc_ehp(eaT, W_e1, b_e1):
    grid = E // EHP_BM
    return pl.pallas_call(
        _ehp_body,
        grid=(grid,),
        in_specs=[
            pl.BlockSpec((16, EHP_BM), lambda i: (0, i)),
            pl.BlockSpec((16, 16), lambda i: (0, 0)),
            pl.BlockSpec((1, 16), lambda i: (0, 0)),
        ],
        out_specs=pl.BlockSpec((EHP_BM // 16, 256), lambda i: (i, 0)),
        out_shape=jax.ShapeDtypeStruct((E // 16, 256), jnp.float32),
        scratch_shapes=[pltpu.VMEM((EHP_BM // 16, 16, 16), jnp.float32)],
    )(eaT, W_e1, b_e1.reshape(1, 16))


def _msg_body(ehp_ref, hsp_ref, bd2_ref, b2t_ref, g_ref, o_ref):
    ehp = ehp_ref[...]
    ew2 = jnp.dot(ehp, bd2_ref[...], preferred_element_type=jnp.float32)
    hs2 = jnp.dot(hsp_ref[...], g_ref[...], preferred_element_type=jnp.float32)
    b2t = b2t_ref[...]
    acc = hs2[:, 0:128] * (ew2[:, 0:128] + b2t[0:1, :])
    for i in range(1, D):
        acc += hs2[:, 128 * i:128 * (i + 1)] * (
            ew2[:, 128 * i:128 * (i + 1)] + b2t[i:i + 1, :]
        )
    o_ref[...] = acc


def _tc_message(ehp, h_src_p, BD2, b2t, G):
    grid = E // MSG_BM
    pb = MSG_BM // 16
    return pl.pallas_call(
        _msg_body,
        grid=(grid,),
        in_specs=[
            pl.BlockSpec((pb, 256), lambda i: (i, 0)),
            pl.BlockSpec((pb, 128), lambda i: (i, 0)),
            pl.BlockSpec((256, D * 128), lambda i: (0, 0)),
            pl.BlockSpec((D, 128), lambda i: (0, 0)),
            pl.BlockSpec((128, D * 128), lambda i: (0, 0)),
        ],
        out_specs=pl.BlockSpec((pb, 128), lambda i: (i, 0)),
        out_shape=jax.ShapeDtypeStruct((E // 16, 128), jnp.float32),
    )(ehp, h_src_p, BD2, b2t, G)


NP = N // 16


def _gru_body(
    agg2_ref, h_ref, hid_ref, wroot_ref, bconv_ref,
    wir_ref, wiz_ref, win_ref, bir_ref, biz_ref, bin_ref,
    whr_ref, whz_ref, whn_ref, bhr_ref, bhz_ref, bhn_ref,
    o_ref,
):
    agg = agg2_ref[0] + agg2_ref[1]
    h = h_ref[...]
    hidden = hid_ref[...]
    m = jnp.maximum(
        agg
        + jnp.dot(h, wroot_ref[...], preferred_element_type=jnp.float32)
        + bconv_ref[...],
        0.0,
    )
    i_r = jnp.dot(m, wir_ref[...], preferred_element_type=jnp.float32) + bir_ref[...]
    i_z = jnp.dot(m, wiz_ref[...], preferred_element_type=jnp.float32) + biz_ref[...]
    i_n = jnp.dot(m, win_ref[...], preferred_element_type=jnp.float32) + bin_ref[...]
    h_r = jnp.dot(hidden, whr_ref[...], preferred_element_type=jnp.float32) + bhr_ref[...]
    h_z = jnp.dot(hidden, whz_ref[...], preferred_element_type=jnp.float32) + bhz_ref[...]
    h_n = jnp.dot(hidden, whn_ref[...], preferred_element_type=jnp.float32) + bhn_ref[...]
    r = jax.nn.sigmoid(i_r + h_r)
    z = jax.nn.sigmoid(i_z + h_z)
    n = jnp.tanh(i_n + r * h_n)
    o_ref[...] = (1.0 - z) * n + z * hidden


def _tc_gru(agg2p, hp, hiddenp, kron_w):
    return pl.pallas_call(
        _gru_body,
        out_shape=jax.ShapeDtypeStruct((NP, 128), jnp.float32),
    )(agg2p, hp, hiddenp, *kron_w)


def _readout_body(
    hp_ref, batchp_ref, kr1_ref, br1_ref, kr2_ref, br2_ref, wp_ref, bp_ref, o_ref
):
    hp = hp_ref[...]
    nfp = jnp.maximum(
        jnp.dot(hp, kr1_ref[...], preferred_element_type=jnp.float32)
        + br1_ref[...],
        0.0,
    )
    nfp = jnp.dot(nfp, kr2_ref[...], preferred_element_type=jnp.float32) + br2_ref[...]
    batchp = batchp_ref[...]
    gid = lax.broadcasted_iota(jnp.int32, (1, NG), 1)
    sums = jnp.zeros((NG, D), jnp.float32)
    oh_sum = jnp.zeros((NP, NG), jnp.float32)
    for k in range(16):
        ohk = (batchp[:, k:k + 1] == gid).astype(jnp.float32)
        oh_sum = oh_sum + ohk
        sk = lax.dot_general(
            ohk, nfp, (((0,), (0,)), ((), ())), preferred_element_type=jnp.float32
        )
        sums = sums + sk[:, D * k:D * (k + 1)]
    counts = lax.dot_general(
        oh_sum, jnp.ones((NP, 1), jnp.float32), (((0,), (0,)), ((), ())),
        preferred_element_type=jnp.float32,
    )
    g = sums / jnp.maximum(counts, 1.0)
    o_ref[...] = (
        jnp.dot(g, wp_ref[...], preferred_element_type=jnp.float32) + bp_ref[...]
    )


def _tc_readout(hp, batchp, W_r1, b_r1, W_r2, b_r2, W_p, b_p):
    eye16 = jnp.eye(16, dtype=jnp.float32)
    return pl.pallas_call(
        _readout_body,
        out_shape=jax.ShapeDtypeStruct((NG, 1), jnp.float32),
    )(hp, batchp, jnp.kron(eye16, W_r1), jnp.tile(b_r1, 16).reshape(1, 128),
      jnp.kron(eye16, W_r2), jnp.tile(b_r2, 16).reshape(1, 128),
      W_p, b_p.reshape(1, 1))


def kernel(x, edge_index, edge_attr, batch,
           W_proj, b_proj, W_e1, b_e1, W_e2, b_e2, W_root, b_conv,
           W_gru_ih, b_gru_ih, W_gru_hh, b_gru_hh,
           W_r1, b_r1, W_r2, b_r2, W_p, b_p):
    src3 = edge_index[0].reshape(NW, NCH, CW)
    dst3 = edge_index[1].reshape(NW, NCH, CW)
    batch2d = batch.reshape(N, 1)
    zeros_nd = jnp.zeros((N, D), jnp.float32)

    eye16 = jnp.eye(16, dtype=jnp.float32)
    BD2 = jnp.concatenate(
        [jnp.kron(eye16, W_e2[:, D * i:D * (i + 1)]) for i in range(D)], axis=1
    )
    b2t = jnp.stack([jnp.tile(b_e2[D * i:D * (i + 1)], 16) for i in range(D)])
    onehot8 = jnp.eye(D, dtype=jnp.float32)
    G = jnp.concatenate(
        [jnp.kron(eye16, onehot8[:, i:i + 1] * jnp.ones((1, D), jnp.float32))
         for i in range(D)], axis=1
    )

    def kt(w):
        return jnp.kron(eye16, w)

    def bt(b):
        return jnp.tile(b, 16).reshape(1, 128)

    kron_w = (
        kt(W_root), bt(b_conv),
        kt(W_gru_ih[:, 0:D]), kt(W_gru_ih[:, D:2 * D]), kt(W_gru_ih[:, 2 * D:]),
        bt(b_gru_ih[0:D]), bt(b_gru_ih[D:2 * D]), bt(b_gru_ih[2 * D:]),
        kt(W_gru_hh[:, 0:D]), kt(W_gru_hh[:, D:2 * D]), kt(W_gru_hh[:, 2 * D:]),
        bt(b_gru_hh[0:D]), bt(b_gru_hh[D:2 * D]), bt(b_gru_hh[2 * D:]),
    )

    ehp = _tc_ehp(edge_attr.T, W_e1, b_e1)
    hp = _tc_project(x, W_proj, b_proj).reshape(NP, 128)
    hiddenp = hp
    for _ in range(STEPS):
        h_src = _sc_gather(hp.reshape(N, D), src3)
        msg_p = _tc_message(ehp, h_src.reshape(E // 16, 128),
                            BD2, b2t, G)
        agg2 = _sc_scatter_add(msg_p.reshape(E, D), dst3, zeros_nd)
        hiddenp = _tc_gru(agg2.reshape(NC, NP, 128), hp, hiddenp, kron_w)
        hp = hiddenp
    return _tc_readout(hp, batch.reshape(NP, 16), W_r1, b_r1, W_r2, b_r2, W_p, b_p)


# speedup vs baseline: 4.6599x; 1.0517x over previous
import functools

import jax
import jax.numpy as jnp
from jax import lax
from jax.experimental import pallas as pl
from jax.experimental.pallas import tpu as pltpu
from jax.experimental.pallas import tpu_sc as plsc

N = 10000
E = 320000
D = 8
NG = 64
STEPS = 3

NC = 2
NS = 16
NW = NC * NS
EPW = E // NW
CW = 125
SUB = 8
SCW = CW * SUB
NSC = EPW // SCW
NCH = EPW // CW
ROWS_PER_TILE = N // NS

_SC_PARAMS = pltpu.CompilerParams(use_tc_tiling_on_sc=False)


@functools.cache
def _sc_mesh():
    return plsc.VectorSubcoreMesh(
        core_axis_name="c", subcore_axis_name="s", num_cores=NC, num_subcores=NS
    )


WAVE = 16


def _gather_body(h_hbm, src_hbm, out_hbm, idx_v, rows_v, gsem, ssem):
    wid = lax.axis_index("s") * NC + lax.axis_index("c")
    base = wid * EPW
    pltpu.sync_copy(src_hbm.at[wid], idx_v)

    def gdesc(j):
        return pltpu.make_async_copy(
            h_hbm.at[idx_v.at[j]], rows_v.at[pl.ds(j * CW, CW)], gsem
        )

    def body(j, carry):
        @pl.when(j < NCH)
        def _():
            gdesc(j).start()

        @pl.when(j >= WAVE)
        def _():
            gdesc(j - WAVE).wait()

        return carry

    lax.fori_loop(0, NCH + WAVE, body, 0)
    pltpu.sync_copy(rows_v, out_hbm.at[pl.ds(base, EPW)])


def _sc_gather(h, src3):
    kern = pl.kernel(
        _gather_body,
        out_type=jax.ShapeDtypeStruct((E, D), jnp.float32),
        mesh=_sc_mesh(),
        scratch_types=[
            pltpu.VMEM((NCH, CW), jnp.int32),
            pltpu.VMEM((EPW, D), jnp.float32),
            pltpu.SemaphoreType.DMA,
            pltpu.SemaphoreType.DMA,
        ],
        compiler_params=_SC_PARAMS,
    )
    return kern(h, src3)


def _scatter_body(msg_hbm, dst_hbm, zeros_hbm, out_hbm, didx_v, mrows_v, agg_sh,
                  lsem, asem):
    cid = lax.axis_index("c")
    sid = lax.axis_index("s")
    wid = sid * NC + cid
    base = wid * EPW

    pltpu.sync_copy(
        zeros_hbm.at[pl.ds(sid * ROWS_PER_TILE, ROWS_PER_TILE)],
        agg_sh.at[pl.ds(sid * ROWS_PER_TILE, ROWS_PER_TILE)],
    )
    mload = pltpu.make_async_copy(
        msg_hbm.at[pl.ds(base, EPW)], mrows_v, lsem
    )
    mload.start()
    pltpu.sync_copy(dst_hbm.at[wid], didx_v)
    plsc.subcore_barrier()
    mload.wait()

    def sdesc(j):
        return pltpu.make_async_copy(
            mrows_v.at[pl.ds(j * CW, CW)], agg_sh.at[didx_v.at[j]], asem
        )

    def body(j, carry):
        @pl.when(j < NCH)
        def _():
            pltpu.async_copy(
                mrows_v.at[pl.ds(j * CW, CW)], agg_sh.at[didx_v.at[j]],
                asem, add=True,
            )

        @pl.when(j >= WAVE)
        def _():
            sdesc(j - WAVE).wait()

        return carry

    lax.fori_loop(0, NCH + WAVE, body, 0)
    plsc.subcore_barrier()

    pltpu.sync_copy(
        agg_sh.at[pl.ds(sid * ROWS_PER_TILE, ROWS_PER_TILE)],
        out_hbm.at[cid, pl.ds(sid * ROWS_PER_TILE, ROWS_PER_TILE)],
    )


def _sc_scatter_add(msg, dst3, zeros_nd):
    kern = pl.kernel(
        _scatter_body,
        out_type=jax.ShapeDtypeStruct((NC, N, D), jnp.float32),
        mesh=_sc_mesh(),
        scratch_types=[
            pltpu.VMEM((NCH, CW), jnp.int32),
            pltpu.VMEM((EPW, D), jnp.float32),
            pltpu.VMEM_SHARED((N, D), jnp.float32),
            pltpu.SemaphoreType.DMA,
            pltpu.SemaphoreType.DMA,
        ],
        compiler_params=_SC_PARAMS,
    )
    return kern(msg, dst3, zeros_nd)


def _proj_body(x_ref, w_ref, b_ref, o_ref):
    o_ref[...] = jnp.maximum(
        jnp.dot(x_ref[...], w_ref[...], preferred_element_type=jnp.float32)
        + b_ref[...],
        0.0,
    )


def _tc_project(x, W_proj, b_proj):
    return pl.pallas_call(
        _proj_body,
        out_shape=jax.ShapeDtypeStruct((N, D), jnp.float32),
    )(x, W_proj, b_proj.reshape(1, D))


MSG_BM = 16000


EHP_BM = 6400


def _ehp_body(eat_ref, we1_ref, be1_ref, o_ref, s3_ref):
    eh = jnp.maximum(
        lax.dot_general(
            eat_ref[...], we1_ref[...], (((0,), (0,)), ((), ())),
            preferred_element_type=jnp.float32,
        )
        + be1_ref[...],
        0.0,
    )
    s3_ref[...] = eh.reshape(EHP_BM // 16, 16, 16)
    o_ref[...] = s3_ref[...].reshape(EHP_BM // 16, 256)


def _tc_ehp(eaT, W_e1, b_e1):
    grid = E // EHP_BM
    return pl.pallas_call(
        _ehp_body,
        grid=(grid,),
        in_specs=[
            pl.BlockSpec((16, EHP_BM), lambda i: (0, i)),
            pl.BlockSpec((16, 16), lambda i: (0, 0)),
            pl.BlockSpec((1, 16), lambda i: (0, 0)),
        ],
        out_specs=pl.BlockSpec((EHP_BM // 16, 256), lambda i: (i, 0)),
        out_shape=jax.ShapeDtypeStruct((E // 16, 256), jnp.float32),
        scratch_shapes=[pltpu.VMEM((EHP_BM // 16, 16, 16), jnp.float32)],
    )(eaT, W_e1, b_e1.reshape(1, 16))


def _msg_body(ehp_ref, hsp_ref, bd2_ref, b2t_ref, g_ref, o_ref):
    ehp = ehp_ref[...]
    ew2 = jnp.dot(ehp, bd2_ref[...], preferred_element_type=jnp.float32)
    hs2 = jnp.dot(hsp_ref[...], g_ref[...], preferred_element_type=jnp.float32)
    b2t = b2t_ref[...]
    acc = hs2[:, 0:128] * (ew2[:, 0:128] + b2t[0:1, :])
    for i in range(1, D):
        acc += hs2[:, 128 * i:128 * (i + 1)] * (
            ew2[:, 128 * i:128 * (i + 1)] + b2t[i:i + 1, :]
        )
    o_ref[...] = acc


def _tc_message(ehp, h_src_p, BD2, b2t, G):
    grid = E // MSG_BM
    pb = MSG_BM // 16
    return pl.pallas_call(
        _msg_body,
        grid=(grid,),
        in_specs=[
            pl.BlockSpec((pb, 256), lambda i: (i, 0)),
            pl.BlockSpec((pb, 128), lambda i: (i, 0)),
            pl.BlockSpec((256, D * 128), lambda i: (0, 0)),
            pl.BlockSpec((D, 128), lambda i: (0, 0)),
            pl.BlockSpec((128, D * 128), lambda i: (0, 0)),
        ],
        out_specs=pl.BlockSpec((pb, 128), lambda i: (i, 0)),
        out_shape=jax.ShapeDtypeStruct((E // 16, 128), jnp.float32),
    )(ehp, h_src_p, BD2, b2t, G)


NP = N // 16


def _gru_body(
    agg2_ref, h_ref, hid_ref, wroot_ref, bconv_ref,
    wir_ref, wiz_ref, win_ref, bir_ref, biz_ref, bin_ref,
    whr_ref, whz_ref, whn_ref, bhr_ref, bhz_ref, bhn_ref,
    o_ref,
):
    agg = agg2_ref[0] + agg2_ref[1]
    h = h_ref[...]
    hidden = hid_ref[...]
    m = jnp.maximum(
        agg
        + jnp.dot(h, wroot_ref[...], preferred_element_type=jnp.float32)
        + bconv_ref[...],
        0.0,
    )
    i_r = jnp.dot(m, wir_ref[...], preferred_element_type=jnp.float32) + bir_ref[...]
    i_z = jnp.dot(m, wiz_ref[...], preferred_element_type=jnp.float32) + biz_ref[...]
    i_n = jnp.dot(m, win_ref[...], preferred_element_type=jnp.float32) + bin_ref[...]
    h_r = jnp.dot(hidden, whr_ref[...], preferred_element_type=jnp.float32) + bhr_ref[...]
    h_z = jnp.dot(hidden, whz_ref[...], preferred_element_type=jnp.float32) + bhz_ref[...]
    h_n = jnp.dot(hidden, whn_ref[...], preferred_element_type=jnp.float32) + bhn_ref[...]
    r = jax.nn.sigmoid(i_r + h_r)
    z = jax.nn.sigmoid(i_z + h_z)
    n = jnp.tanh(i_n + r * h_n)
    o_ref[...] = (1.0 - z) * n + z * hidden


def _tc_gru(agg2p, hp, hiddenp, kron_w):
    return pl.pallas_call(
        _gru_body,
        out_shape=jax.ShapeDtypeStruct((NP, 128), jnp.float32),
    )(agg2p, hp, hiddenp, *kron_w)


def _readout_body(
    hp_ref, batchp_ref, kr1_ref, br1_ref, kr2_ref, br2_ref, wp_ref, bp_ref, o_ref
):
    hp = hp_ref[...]
    nfp = jnp.maximum(
        jnp.dot(hp, kr1_ref[...], preferred_element_type=jnp.float32)
        + br1_ref[...],
        0.0,
    )
    nfp = jnp.dot(nfp, kr2_ref[...], preferred_element_type=jnp.float32) + br2_ref[...]
    batchp = batchp_ref[...]
    gid = lax.broadcasted_iota(jnp.int32, (1, NG), 1)
    sums = jnp.zeros((NG, D), jnp.float32)
    oh_sum = jnp.zeros((NP, NG), jnp.float32)
    for k in range(16):
        ohk = (batchp[:, k:k + 1] == gid).astype(jnp.float32)
        oh_sum = oh_sum + ohk
        sk = lax.dot_general(
            ohk, nfp, (((0,), (0,)), ((), ())), preferred_element_type=jnp.float32
        )
        sums = sums + sk[:, D * k:D * (k + 1)]
    counts = lax.dot_general(
        oh_sum, jnp.ones((NP, 1), jnp.float32), (((0,), (0,)), ((), ())),
        preferred_element_type=jnp.float32,
    )
    g = sums / jnp.maximum(counts, 1.0)
    o_ref[...] = (
        jnp.dot(g, wp_ref[...], preferred_element_type=jnp.float32) + bp_ref[...]
    )


def _tc_readout(hp, batchp, W_r1, b_r1, W_r2, b_r2, W_p, b_p):
    eye16 = jnp.eye(16, dtype=jnp.float32)
    return pl.pallas_call(
        _readout_body,
        out_shape=jax.ShapeDtypeStruct((NG, 1), jnp.float32),
    )(hp, batchp, jnp.kron(eye16, W_r1), jnp.tile(b_r1, 16).reshape(1, 128),
      jnp.kron(eye16, W_r2), jnp.tile(b_r2, 16).reshape(1, 128),
      W_p, b_p.reshape(1, 1))


def kernel(x, edge_index, edge_attr, batch,
           W_proj, b_proj, W_e1, b_e1, W_e2, b_e2, W_root, b_conv,
           W_gru_ih, b_gru_ih, W_gru_hh, b_gru_hh,
           W_r1, b_r1, W_r2, b_r2, W_p, b_p):
    src3 = edge_index[0].reshape(NW, NCH, CW)
    dst3 = edge_index[1].reshape(NW, NCH, CW)
    batch2d = batch.reshape(N, 1)
    zeros_nd = jnp.zeros((N, D), jnp.float32)

    eye16 = jnp.eye(16, dtype=jnp.float32)
    BD2 = jnp.concatenate(
        [jnp.kron(eye16, W_e2[:, D * i:D * (i + 1)]) for i in range(D)], axis=1
    )
    b2t = jnp.stack([jnp.tile(b_e2[D * i:D * (i + 1)], 16) for i in range(D)])
    onehot8 = jnp.eye(D, dtype=jnp.float32)
    G = jnp.concatenate(
        [jnp.kron(eye16, onehot8[:, i:i + 1] * jnp.ones((1, D), jnp.float32))
         for i in range(D)], axis=1
    )

    def kt(w):
        return jnp.kron(eye16, w)

    def bt(b):
        return jnp.tile(b, 16).reshape(1, 128)

    kron_w = (
        kt(W_root), bt(b_conv),
        kt(W_gru_ih[:, 0:D]), kt(W_gru_ih[:, D:2 * D]), kt(W_gru_ih[:, 2 * D:]),
        bt(b_gru_ih[0:D]), bt(b_gru_ih[D:2 * D]), bt(b_gru_ih[2 * D:]),
        kt(W_gru_hh[:, 0:D]), kt(W_gru_hh[:, D:2 * D]), kt(W_gru_hh[:, 2 * D:]),
        bt(b_gru_hh[0:D]), bt(b_gru_hh[D:2 * D]), bt(b_gru_hh[2 * D:]),
    )

    ehp = _tc_ehp(edge_attr.T, W_e1, b_e1)
    hp = _tc_project(x, W_proj, b_proj).reshape(NP, 128)
    hiddenp = hp
    for _ in range(STEPS):
        h_src = _sc_gather(hp.reshape(N, D), src3)
        msg_p = _tc_message(ehp, h_src.reshape(E // 16, 128),
                            BD2, b2t, G)
        agg2 = _sc_scatter_add(msg_p.reshape(E, D), dst3, zeros_nd)
        hiddenp = _tc_gru(agg2.reshape(NC, NP, 128), hp, hiddenp, kron_w)
        hp = hiddenp
    return _tc_readout(hp, batch.reshape(NP, 16), W_r1, b_r1, W_r2, b_r2, W_p, b_p)


# speedup vs baseline: 4.8396x; 1.0386x over previous
import functools

import jax
import jax.numpy as jnp
from jax import lax
from jax.experimental import pallas as pl
from jax.experimental.pallas import tpu as pltpu
from jax.experimental.pallas import tpu_sc as plsc

N = 10000
E = 320000
D = 8
NG = 64
STEPS = 3

NC = 2
NS = 16
NW = NC * NS
EPW = E // NW
CW = 125
SUB = 8
SCW = CW * SUB
NSC = EPW // SCW
NCH = EPW // CW
ROWS_PER_TILE = N // NS

_SC_PARAMS = pltpu.CompilerParams(use_tc_tiling_on_sc=False)


@functools.cache
def _sc_mesh():
    return plsc.VectorSubcoreMesh(
        core_axis_name="c", subcore_axis_name="s", num_cores=NC, num_subcores=NS
    )


WAVE = 32


def _gather_body(h_hbm, src_hbm, out_hbm, idx_v, rows_v, gsem, ssem):
    wid = lax.axis_index("s") * NC + lax.axis_index("c")
    base = wid * EPW
    pltpu.sync_copy(src_hbm.at[wid], idx_v)

    def gdesc(j):
        return pltpu.make_async_copy(
            h_hbm.at[idx_v.at[j]], rows_v.at[pl.ds(j * CW, CW)], gsem
        )

    def body(j, carry):
        @pl.when(j < NCH)
        def _():
            gdesc(j).start()

        @pl.when(j >= WAVE)
        def _():
            gdesc(j - WAVE).wait()

        return carry

    lax.fori_loop(0, NCH + WAVE, body, 0)
    pltpu.sync_copy(rows_v, out_hbm.at[pl.ds(base, EPW)])


def _sc_gather(h, src3):
    kern = pl.kernel(
        _gather_body,
        out_type=jax.ShapeDtypeStruct((E, D), jnp.float32),
        mesh=_sc_mesh(),
        scratch_types=[
            pltpu.VMEM((NCH, CW), jnp.int32),
            pltpu.VMEM((EPW, D), jnp.float32),
            pltpu.SemaphoreType.DMA,
            pltpu.SemaphoreType.DMA,
        ],
        compiler_params=_SC_PARAMS,
    )
    return kern(h, src3)


def _scatter_body(msg_hbm, dst_hbm, zeros_hbm, out_hbm, didx_v, mrows_v, agg_sh,
                  lsem, asem):
    cid = lax.axis_index("c")
    sid = lax.axis_index("s")
    wid = sid * NC + cid
    base = wid * EPW

    pltpu.sync_copy(
        zeros_hbm.at[pl.ds(sid * ROWS_PER_TILE, ROWS_PER_TILE)],
        agg_sh.at[pl.ds(sid * ROWS_PER_TILE, ROWS_PER_TILE)],
    )
    mload = pltpu.make_async_copy(
        msg_hbm.at[pl.ds(base, EPW)], mrows_v, lsem
    )
    mload.start()
    pltpu.sync_copy(dst_hbm.at[wid], didx_v)
    plsc.subcore_barrier()
    mload.wait()

    def sdesc(j):
        return pltpu.make_async_copy(
            mrows_v.at[pl.ds(j * CW, CW)], agg_sh.at[didx_v.at[j]], asem
        )

    def body(j, carry):
        @pl.when(j < NCH)
        def _():
            pltpu.async_copy(
                mrows_v.at[pl.ds(j * CW, CW)], agg_sh.at[didx_v.at[j]],
                asem, add=True,
            )

        @pl.when(j >= WAVE)
        def _():
            sdesc(j - WAVE).wait()

        return carry

    lax.fori_loop(0, NCH + WAVE, body, 0)
    plsc.subcore_barrier()

    pltpu.sync_copy(
        agg_sh.at[pl.ds(sid * ROWS_PER_TILE, ROWS_PER_TILE)],
        out_hbm.at[cid, pl.ds(sid * ROWS_PER_TILE, ROWS_PER_TILE)],
    )


def _sc_scatter_add(msg, dst3, zeros_nd):
    kern = pl.kernel(
        _scatter_body,
        out_type=jax.ShapeDtypeStruct((NC, N, D), jnp.float32),
        mesh=_sc_mesh(),
        scratch_types=[
            pltpu.VMEM((NCH, CW), jnp.int32),
            pltpu.VMEM((EPW, D), jnp.float32),
            pltpu.VMEM_SHARED((N, D), jnp.float32),
            pltpu.SemaphoreType.DMA,
            pltpu.SemaphoreType.DMA,
        ],
        compiler_params=_SC_PARAMS,
    )
    return kern(msg, dst3, zeros_nd)


def _proj_body(x_ref, w_ref, b_ref, o_ref):
    o_ref[...] = jnp.maximum(
        jnp.dot(x_ref[...], w_ref[...], preferred_element_type=jnp.float32)
        + b_ref[...],
        0.0,
    )


def _tc_project(x, W_proj, b_proj):
    return pl.pallas_call(
        _proj_body,
        out_shape=jax.ShapeDtypeStruct((N, D), jnp.float32),
    )(x, W_proj, b_proj.reshape(1, D))


MSG_BM = 32000


EHP_BM = 6400


def _ehp_body(eat_ref, we1_ref, be1_ref, o_ref, s3_ref):
    eh = jnp.maximum(
        lax.dot_general(
            eat_ref[...], we1_ref[...], (((0,), (0,)), ((), ())),
            preferred_element_type=jnp.float32,
        )
        + be1_ref[...],
        0.0,
    )
    s3_ref[...] = eh.reshape(EHP_BM // 16, 16, 16)
    o_ref[...] = s3_ref[...].reshape(EHP_BM // 16, 256)


def _tc_ehp(eaT, W_e1, b_e1):
    grid = E // EHP_BM
    return pl.pallas_call(
        _ehp_body,
        grid=(grid,),
        in_specs=[
            pl.BlockSpec((16, EHP_BM), lambda i: (0, i)),
            pl.BlockSpec((16, 16), lambda i: (0, 0)),
            pl.BlockSpec((1, 16), lambda i: (0, 0)),
        ],
        out_specs=pl.BlockSpec((EHP_BM // 16, 256), lambda i: (i, 0)),
        out_shape=jax.ShapeDtypeStruct((E // 16, 256), jnp.float32),
        scratch_shapes=[pltpu.VMEM((EHP_BM // 16, 16, 16), jnp.float32)],
    )(eaT, W_e1, b_e1.reshape(1, 16))


def _msg_body(ehp_ref, hsp_ref, bd2_ref, b2t_ref, g_ref, o_ref):
    ehp = ehp_ref[...]
    ew2 = jnp.dot(ehp, bd2_ref[...], preferred_element_type=jnp.float32)
    hs2 = jnp.dot(hsp_ref[...], g_ref[...], preferred_element_type=jnp.float32)
    b2t = b2t_ref[...]
    acc = hs2[:, 0:128] * (ew2[:, 0:128] + b2t[0:1, :])
    for i in range(1, D):
        acc += hs2[:, 128 * i:128 * (i + 1)] * (
            ew2[:, 128 * i:128 * (i + 1)] + b2t[i:i + 1, :]
        )
    o_ref[...] = acc


def _tc_message(ehp, h_src_p, BD2, b2t, G):
    grid = E // MSG_BM
    pb = MSG_BM // 16
    return pl.pallas_call(
        _msg_body,
        grid=(grid,),
        in_specs=[
            pl.BlockSpec((pb, 256), lambda i: (i, 0)),
            pl.BlockSpec((pb, 128), lambda i: (i, 0)),
            pl.BlockSpec((256, D * 128), lambda i: (0, 0)),
            pl.BlockSpec((D, 128), lambda i: (0, 0)),
            pl.BlockSpec((128, D * 128), lambda i: (0, 0)),
        ],
        out_specs=pl.BlockSpec((pb, 128), lambda i: (i, 0)),
        out_shape=jax.ShapeDtypeStruct((E // 16, 128), jnp.float32),
    )(ehp, h_src_p, BD2, b2t, G)


NP = N // 16


def _gru_body(
    agg2_ref, h_ref, hid_ref, wroot_ref, bconv_ref,
    wir_ref, wiz_ref, win_ref, bir_ref, biz_ref, bin_ref,
    whr_ref, whz_ref, whn_ref, bhr_ref, bhz_ref, bhn_ref,
    o_ref,
):
    agg = agg2_ref[0] + agg2_ref[1]
    h = h_ref[...]
    hidden = hid_ref[...]
    m = jnp.maximum(
        agg
        + jnp.dot(h, wroot_ref[...], preferred_element_type=jnp.float32)
        + bconv_ref[...],
        0.0,
    )
    i_r = jnp.dot(m, wir_ref[...], preferred_element_type=jnp.float32) + bir_ref[...]
    i_z = jnp.dot(m, wiz_ref[...], preferred_element_type=jnp.float32) + biz_ref[...]
    i_n = jnp.dot(m, win_ref[...], preferred_element_type=jnp.float32) + bin_ref[...]
    h_r = jnp.dot(hidden, whr_ref[...], preferred_element_type=jnp.float32) + bhr_ref[...]
    h_z = jnp.dot(hidden, whz_ref[...], preferred_element_type=jnp.float32) + bhz_ref[...]
    h_n = jnp.dot(hidden, whn_ref[...], preferred_element_type=jnp.float32) + bhn_ref[...]
    r = jax.nn.sigmoid(i_r + h_r)
    z = jax.nn.sigmoid(i_z + h_z)
    n = jnp.tanh(i_n + r * h_n)
    o_ref[...] = (1.0 - z) * n + z * hidden


def _tc_gru(agg2p, hp, hiddenp, kron_w):
    return pl.pallas_call(
        _gru_body,
        out_shape=jax.ShapeDtypeStruct((NP, 128), jnp.float32),
    )(agg2p, hp, hiddenp, *kron_w)


def _readout_body(
    hp_ref, batchp_ref, kr1_ref, br1_ref, kr2_ref, br2_ref, wp_ref, bp_ref, o_ref
):
    hp = hp_ref[...]
    nfp = jnp.maximum(
        jnp.dot(hp, kr1_ref[...], preferred_element_type=jnp.float32)
        + br1_ref[...],
        0.0,
    )
    nfp = jnp.dot(nfp, kr2_ref[...], preferred_element_type=jnp.float32) + br2_ref[...]
    batchp = batchp_ref[...]
    gid = lax.broadcasted_iota(jnp.int32, (1, NG), 1)
    sums = jnp.zeros((NG, D), jnp.float32)
    oh_sum = jnp.zeros((NP, NG), jnp.float32)
    for k in range(16):
        ohk = (batchp[:, k:k + 1] == gid).astype(jnp.float32)
        oh_sum = oh_sum + ohk
        sk = lax.dot_general(
            ohk, nfp, (((0,), (0,)), ((), ())), preferred_element_type=jnp.float32
        )
        sums = sums + sk[:, D * k:D * (k + 1)]
    counts = lax.dot_general(
        oh_sum, jnp.ones((NP, 1), jnp.float32), (((0,), (0,)), ((), ())),
        preferred_element_type=jnp.float32,
    )
    g = sums / jnp.maximum(counts, 1.0)
    o_ref[...] = (
        jnp.dot(g, wp_ref[...], preferred_element_type=jnp.float32) + bp_ref[...]
    )


def _tc_readout(hp, batchp, W_r1, b_r1, W_r2, b_r2, W_p, b_p):
    eye16 = jnp.eye(16, dtype=jnp.float32)
    return pl.pallas_call(
        _readout_body,
        out_shape=jax.ShapeDtypeStruct((NG, 1), jnp.float32),
    )(hp, batchp, jnp.kron(eye16, W_r1), jnp.tile(b_r1, 16).reshape(1, 128),
      jnp.kron(eye16, W_r2), jnp.tile(b_r2, 16).reshape(1, 128),
      W_p, b_p.reshape(1, 1))


def kernel(x, edge_index, edge_attr, batch,
           W_proj, b_proj, W_e1, b_e1, W_e2, b_e2, W_root, b_conv,
           W_gru_ih, b_gru_ih, W_gru_hh, b_gru_hh,
           W_r1, b_r1, W_r2, b_r2, W_p, b_p):
    src3 = edge_index[0].reshape(NW, NCH, CW)
    dst3 = edge_index[1].reshape(NW, NCH, CW)
    batch2d = batch.reshape(N, 1)
    zeros_nd = jnp.zeros((N, D), jnp.float32)

    eye16 = jnp.eye(16, dtype=jnp.float32)
    BD2 = jnp.concatenate(
        [jnp.kron(eye16, W_e2[:, D * i:D * (i + 1)]) for i in range(D)], axis=1
    )
    b2t = jnp.stack([jnp.tile(b_e2[D * i:D * (i + 1)], 16) for i in range(D)])
    onehot8 = jnp.eye(D, dtype=jnp.float32)
    G = jnp.concatenate(
        [jnp.kron(eye16, onehot8[:, i:i + 1] * jnp.ones((1, D), jnp.float32))
         for i in range(D)], axis=1
    )

    def kt(w):
        return jnp.kron(eye16, w)

    def bt(b):
        return jnp.tile(b, 16).reshape(1, 128)

    kron_w = (
        kt(W_root), bt(b_conv),
        kt(W_gru_ih[:, 0:D]), kt(W_gru_ih[:, D:2 * D]), kt(W_gru_ih[:, 2 * D:]),
        bt(b_gru_ih[0:D]), bt(b_gru_ih[D:2 * D]), bt(b_gru_ih[2 * D:]),
        kt(W_gru_hh[:, 0:D]), kt(W_gru_hh[:, D:2 * D]), kt(W_gru_hh[:, 2 * D:]),
        bt(b_gru_hh[0:D]), bt(b_gru_hh[D:2 * D]), bt(b_gru_hh[2 * D:]),
    )

    ehp = _tc_ehp(edge_attr.T, W_e1, b_e1)
    hp = _tc_project(x, W_proj, b_proj).reshape(NP, 128)
    hiddenp = hp
    for _ in range(STEPS):
        h_src = _sc_gather(hp.reshape(N, D), src3)
        msg_p = _tc_message(ehp, h_src.reshape(E // 16, 128),
                            BD2, b2t, G)
        agg2 = _sc_scatter_add(msg_p.reshape(E, D), dst3, zeros_nd)
        hiddenp = _tc_gru(agg2.reshape(NC, NP, 128), hp, hiddenp, kron_w)
        hp = hiddenp
    return _tc_readout(hp, batch.reshape(NP, 16), W_r1, b_r1, W_r2, b_r2, W_p, b_p)


# speedup vs baseline: 4.8988x; 1.0122x over previous
import functools

import jax
import jax.numpy as jnp
from jax import lax
from jax.experimental import pallas as pl
from jax.experimental.pallas import tpu as pltpu
from jax.experimental.pallas import tpu_sc as plsc

N = 10000
E = 320000
D = 8
NG = 64
STEPS = 3

NC = 2
NS = 16
NW = NC * NS
EPW = E // NW
CW = 125
SUB = 8
SCW = CW * SUB
NSC = EPW // SCW
NCH = EPW // CW
ROWS_PER_TILE = N // NS

_SC_PARAMS = pltpu.CompilerParams(use_tc_tiling_on_sc=False)


@functools.cache
def _sc_mesh():
    return plsc.VectorSubcoreMesh(
        core_axis_name="c", subcore_axis_name="s", num_cores=NC, num_subcores=NS
    )


WAVE = 32


def _gather_body(h_hbm, src_hbm, out_hbm, idx_v, rows_v, gsem, ssem):
    wid = lax.axis_index("s") * NC + lax.axis_index("c")
    base = wid * EPW
    pltpu.sync_copy(src_hbm.at[wid], idx_v)

    def gdesc(j):
        return pltpu.make_async_copy(
            h_hbm.at[idx_v.at[j]], rows_v.at[pl.ds(j * CW, CW)], gsem
        )

    def body(j, carry):
        @pl.when(j < NCH)
        def _():
            gdesc(j).start()

        @pl.when(j >= WAVE)
        def _():
            gdesc(j - WAVE).wait()

        return carry

    lax.fori_loop(0, NCH + WAVE, body, 0)
    pltpu.sync_copy(rows_v, out_hbm.at[pl.ds(base, EPW)])


def _sc_gather(h, src3):
    kern = pl.kernel(
        _gather_body,
        out_type=jax.ShapeDtypeStruct((E, D), jnp.float32),
        mesh=_sc_mesh(),
        scratch_types=[
            pltpu.VMEM((NCH, CW), jnp.int32),
            pltpu.VMEM((EPW, D), jnp.float32),
            pltpu.SemaphoreType.DMA,
            pltpu.SemaphoreType.DMA,
        ],
        compiler_params=_SC_PARAMS,
    )
    return kern(h, src3)


def _scatter_body(msg_hbm, dst_hbm, zeros_hbm, out_hbm, didx_v, mrows_v, agg_sh,
                  lsem, asem):
    cid = lax.axis_index("c")
    sid = lax.axis_index("s")
    wid = sid * NC + cid
    base = wid * EPW

    pltpu.sync_copy(
        zeros_hbm.at[pl.ds(sid * ROWS_PER_TILE, ROWS_PER_TILE)],
        agg_sh.at[pl.ds(sid * ROWS_PER_TILE, ROWS_PER_TILE)],
    )
    mload = pltpu.make_async_copy(
        msg_hbm.at[pl.ds(base, EPW)], mrows_v, lsem
    )
    mload.start()
    pltpu.sync_copy(dst_hbm.at[wid], didx_v)
    plsc.subcore_barrier()
    mload.wait()

    def sdesc(j):
        return pltpu.make_async_copy(
            mrows_v.at[pl.ds(j * CW, CW)], agg_sh.at[didx_v.at[j]], asem
        )

    def body(j, carry):
        @pl.when(j < NCH)
        def _():
            pltpu.async_copy(
                mrows_v.at[pl.ds(j * CW, CW)], agg_sh.at[didx_v.at[j]],
                asem, add=True,
            )

        @pl.when(j >= WAVE)
        def _():
            sdesc(j - WAVE).wait()

        return carry

    lax.fori_loop(0, NCH + WAVE, body, 0)
    plsc.subcore_barrier()

    pltpu.sync_copy(
        agg_sh.at[pl.ds(sid * ROWS_PER_TILE, ROWS_PER_TILE)],
        out_hbm.at[cid, pl.ds(sid * ROWS_PER_TILE, ROWS_PER_TILE)],
    )


def _sc_scatter_add(msg, dst3, zeros_nd):
    kern = pl.kernel(
        _scatter_body,
        out_type=jax.ShapeDtypeStruct((NC, N, D), jnp.float32),
        mesh=_sc_mesh(),
        scratch_types=[
            pltpu.VMEM((NCH, CW), jnp.int32),
            pltpu.VMEM((EPW, D), jnp.float32),
            pltpu.VMEM_SHARED((N, D), jnp.float32),
            pltpu.SemaphoreType.DMA,
            pltpu.SemaphoreType.DMA,
        ],
        compiler_params=_SC_PARAMS,
    )
    return kern(msg, dst3, zeros_nd)


def _proj_body(x_ref, w_ref, b_ref, o_ref):
    o_ref[...] = jnp.maximum(
        jnp.dot(x_ref[...], w_ref[...], preferred_element_type=jnp.float32)
        + b_ref[...],
        0.0,
    )


def _tc_project(x, W_proj, b_proj):
    return pl.pallas_call(
        _proj_body,
        out_shape=jax.ShapeDtypeStruct((N, D), jnp.float32),
    )(x, W_proj, b_proj.reshape(1, D))


MSG_BM = 32000


EHP_BM = 6400


def _ehp_body(eat_ref, we1_ref, be1_ref, o_ref, s3_ref):
    eh = jnp.maximum(
        lax.dot_general(
            eat_ref[...], we1_ref[...], (((0,), (0,)), ((), ())),
            preferred_element_type=jnp.float32,
        )
        + be1_ref[...],
        0.0,
    )
    s3_ref[...] = eh.reshape(EHP_BM // 16, 16, 16)
    o_ref[...] = s3_ref[...].reshape(EHP_BM // 16, 256)


def _tc_ehp(eaT, W_e1, b_e1):
    grid = E // EHP_BM
    return pl.pallas_call(
        _ehp_body,
        grid=(grid,),
        in_specs=[
            pl.BlockSpec((16, EHP_BM), lambda i: (0, i)),
            pl.BlockSpec((16, 16), lambda i: (0, 0)),
            pl.BlockSpec((1, 16), lambda i: (0, 0)),
        ],
        out_specs=pl.BlockSpec((EHP_BM // 16, 256), lambda i: (i, 0)),
        out_shape=jax.ShapeDtypeStruct((E // 16, 256), jnp.float32),
        scratch_shapes=[pltpu.VMEM((EHP_BM // 16, 16, 16), jnp.float32)],
    )(eaT, W_e1, b_e1.reshape(1, 16))


def _msg_body(ehp_ref, hsp_ref, bd2_ref, b2t_ref, g_ref, o_ref):
    ehp = ehp_ref[...]
    ew2 = jnp.dot(ehp, bd2_ref[...], preferred_element_type=jnp.float32)
    hs2 = jnp.dot(hsp_ref[...], g_ref[...], preferred_element_type=jnp.float32)
    b2t = b2t_ref[...]
    acc = hs2[:, 0:128] * (ew2[:, 0:128] + b2t[0:1, :])
    for i in range(1, D):
        acc += hs2[:, 128 * i:128 * (i + 1)] * (
            ew2[:, 128 * i:128 * (i + 1)] + b2t[i:i + 1, :]
        )
    o_ref[...] = acc


def _tc_message(ehp, h_src_p, BD2, b2t, G):
    grid = E // MSG_BM
    pb = MSG_BM // 16
    return pl.pallas_call(
        _msg_body,
        grid=(grid,),
        in_specs=[
            pl.BlockSpec((pb, 256), lambda i: (i, 0)),
            pl.BlockSpec((pb, 128), lambda i: (i, 0)),
            pl.BlockSpec((256, D * 128), lambda i: (0, 0)),
            pl.BlockSpec((D, 128), lambda i: (0, 0)),
            pl.BlockSpec((128, D * 128), lambda i: (0, 0)),
        ],
        out_specs=pl.BlockSpec((pb, 128), lambda i: (i, 0)),
        out_shape=jax.ShapeDtypeStruct((E // 16, 128), jnp.float32),
    )(ehp, h_src_p, BD2, b2t, G)


NP = N // 16


def _gru_body(
    agg2_ref, h_ref, hid_ref, wroot_ref, bconv_ref,
    wir_ref, wiz_ref, win_ref, bir_ref, biz_ref, bin_ref,
    whr_ref, whz_ref, whn_ref, bhr_ref, bhz_ref, bhn_ref,
    o_ref,
):
    agg = agg2_ref[0] + agg2_ref[1]
    h = h_ref[...]
    hidden = hid_ref[...]
    m = jnp.maximum(
        agg
        + jnp.dot(h, wroot_ref[...], preferred_element_type=jnp.float32)
        + bconv_ref[...],
        0.0,
    )
    i_r = jnp.dot(m, wir_ref[...], preferred_element_type=jnp.float32) + bir_ref[...]
    i_z = jnp.dot(m, wiz_ref[...], preferred_element_type=jnp.float32) + biz_ref[...]
    i_n = jnp.dot(m, win_ref[...], preferred_element_type=jnp.float32) + bin_ref[...]
    h_r = jnp.dot(hidden, whr_ref[...], preferred_element_type=jnp.float32) + bhr_ref[...]
    h_z = jnp.dot(hidden, whz_ref[...], preferred_element_type=jnp.float32) + bhz_ref[...]
    h_n = jnp.dot(hidden, whn_ref[...], preferred_element_type=jnp.float32) + bhn_ref[...]
    r = jax.nn.sigmoid(i_r + h_r)
    z = jax.nn.sigmoid(i_z + h_z)
    n = jnp.tanh(i_n + r * h_n)
    o_ref[...] = (1.0 - z) * n + z * hidden


def _tc_gru(agg2p, hp, hiddenp, kron_w):
    return pl.pallas_call(
        _gru_body,
        out_shape=jax.ShapeDtypeStruct((NP, 128), jnp.float32),
    )(agg2p, hp, hiddenp, *kron_w)


def _gru_readout_body(
    agg2_ref, h_ref, hid_ref, wroot_ref, bconv_ref,
    wir_ref, wiz_ref, win_ref, bir_ref, biz_ref, bin_ref,
    whr_ref, whz_ref, whn_ref, bhr_ref, bhz_ref, bhn_ref,
    batchp_ref, kr1_ref, br1_ref, kr2_ref, br2_ref, wp_ref, bp_ref,
    o_ref,
):
    agg = agg2_ref[0] + agg2_ref[1]
    h = h_ref[...]
    hidden = hid_ref[...]
    m = jnp.maximum(
        agg
        + jnp.dot(h, wroot_ref[...], preferred_element_type=jnp.float32)
        + bconv_ref[...],
        0.0,
    )
    i_r = jnp.dot(m, wir_ref[...], preferred_element_type=jnp.float32) + bir_ref[...]
    i_z = jnp.dot(m, wiz_ref[...], preferred_element_type=jnp.float32) + biz_ref[...]
    i_n = jnp.dot(m, win_ref[...], preferred_element_type=jnp.float32) + bin_ref[...]
    h_r = jnp.dot(hidden, whr_ref[...], preferred_element_type=jnp.float32) + bhr_ref[...]
    h_z = jnp.dot(hidden, whz_ref[...], preferred_element_type=jnp.float32) + bhz_ref[...]
    h_n = jnp.dot(hidden, whn_ref[...], preferred_element_type=jnp.float32) + bhn_ref[...]
    r = jax.nn.sigmoid(i_r + h_r)
    z = jax.nn.sigmoid(i_z + h_z)
    n = jnp.tanh(i_n + r * h_n)
    hp = (1.0 - z) * n + z * hidden

    nfp = jnp.maximum(
        jnp.dot(hp, kr1_ref[...], preferred_element_type=jnp.float32)
        + br1_ref[...],
        0.0,
    )
    nfp = jnp.dot(nfp, kr2_ref[...], preferred_element_type=jnp.float32) + br2_ref[...]
    batchp = batchp_ref[...]
    gid = lax.broadcasted_iota(jnp.int32, (1, NG), 1)
    sums = jnp.zeros((NG, D), jnp.float32)
    oh_sum = jnp.zeros((NP, NG), jnp.float32)
    for k in range(16):
        ohk = (batchp[:, k:k + 1] == gid).astype(jnp.float32)
        oh_sum = oh_sum + ohk
        sk = lax.dot_general(
            ohk, nfp, (((0,), (0,)), ((), ())), preferred_element_type=jnp.float32
        )
        sums = sums + sk[:, D * k:D * (k + 1)]
    counts = lax.dot_general(
        oh_sum, jnp.ones((NP, 1), jnp.float32), (((0,), (0,)), ((), ())),
        preferred_element_type=jnp.float32,
    )
    g = sums / jnp.maximum(counts, 1.0)
    o_ref[...] = (
        jnp.dot(g, wp_ref[...], preferred_element_type=jnp.float32) + bp_ref[...]
    )


def _tc_gru_readout(agg2p, hp, hiddenp, kron_w, batchp,
                    W_r1, b_r1, W_r2, b_r2, W_p, b_p):
    eye16 = jnp.eye(16, dtype=jnp.float32)
    return pl.pallas_call(
        _gru_readout_body,
        out_shape=jax.ShapeDtypeStruct((NG, 1), jnp.float32),
    )(agg2p, hp, hiddenp, *kron_w, batchp,
      jnp.kron(eye16, W_r1), jnp.tile(b_r1, 16).reshape(1, 128),
      jnp.kron(eye16, W_r2), jnp.tile(b_r2, 16).reshape(1, 128),
      W_p, b_p.reshape(1, 1))


def _readout_body(
    hp_ref, batchp_ref, kr1_ref, br1_ref, kr2_ref, br2_ref, wp_ref, bp_ref, o_ref
):
    hp = hp_ref[...]
    nfp = jnp.maximum(
        jnp.dot(hp, kr1_ref[...], preferred_element_type=jnp.float32)
        + br1_ref[...],
        0.0,
    )
    nfp = jnp.dot(nfp, kr2_ref[...], preferred_element_type=jnp.float32) + br2_ref[...]
    batchp = batchp_ref[...]
    gid = lax.broadcasted_iota(jnp.int32, (1, NG), 1)
    sums = jnp.zeros((NG, D), jnp.float32)
    oh_sum = jnp.zeros((NP, NG), jnp.float32)
    for k in range(16):
        ohk = (batchp[:, k:k + 1] == gid).astype(jnp.float32)
        oh_sum = oh_sum + ohk
        sk = lax.dot_general(
            ohk, nfp, (((0,), (0,)), ((), ())), preferred_element_type=jnp.float32
        )
        sums = sums + sk[:, D * k:D * (k + 1)]
    counts = lax.dot_general(
        oh_sum, jnp.ones((NP, 1), jnp.float32), (((0,), (0,)), ((), ())),
        preferred_element_type=jnp.float32,
    )
    g = sums / jnp.maximum(counts, 1.0)
    o_ref[...] = (
        jnp.dot(g, wp_ref[...], preferred_element_type=jnp.float32) + bp_ref[...]
    )


def _tc_readout(hp, batchp, W_r1, b_r1, W_r2, b_r2, W_p, b_p):
    eye16 = jnp.eye(16, dtype=jnp.float32)
    return pl.pallas_call(
        _readout_body,
        out_shape=jax.ShapeDtypeStruct((NG, 1), jnp.float32),
    )(hp, batchp, jnp.kron(eye16, W_r1), jnp.tile(b_r1, 16).reshape(1, 128),
      jnp.kron(eye16, W_r2), jnp.tile(b_r2, 16).reshape(1, 128),
      W_p, b_p.reshape(1, 1))


def kernel(x, edge_index, edge_attr, batch,
           W_proj, b_proj, W_e1, b_e1, W_e2, b_e2, W_root, b_conv,
           W_gru_ih, b_gru_ih, W_gru_hh, b_gru_hh,
           W_r1, b_r1, W_r2, b_r2, W_p, b_p):
    src3 = edge_index[0].reshape(NW, NCH, CW)
    dst3 = edge_index[1].reshape(NW, NCH, CW)
    batch2d = batch.reshape(N, 1)
    zeros_nd = jnp.zeros((N, D), jnp.float32)

    eye16 = jnp.eye(16, dtype=jnp.float32)
    BD2 = jnp.concatenate(
        [jnp.kron(eye16, W_e2[:, D * i:D * (i + 1)]) for i in range(D)], axis=1
    )
    b2t = jnp.stack([jnp.tile(b_e2[D * i:D * (i + 1)], 16) for i in range(D)])
    onehot8 = jnp.eye(D, dtype=jnp.float32)
    G = jnp.concatenate(
        [jnp.kron(eye16, onehot8[:, i:i + 1] * jnp.ones((1, D), jnp.float32))
         for i in range(D)], axis=1
    )

    def kt(w):
        return jnp.kron(eye16, w)

    def bt(b):
        return jnp.tile(b, 16).reshape(1, 128)

    kron_w = (
        kt(W_root), bt(b_conv),
        kt(W_gru_ih[:, 0:D]), kt(W_gru_ih[:, D:2 * D]), kt(W_gru_ih[:, 2 * D:]),
        bt(b_gru_ih[0:D]), bt(b_gru_ih[D:2 * D]), bt(b_gru_ih[2 * D:]),
        kt(W_gru_hh[:, 0:D]), kt(W_gru_hh[:, D:2 * D]), kt(W_gru_hh[:, 2 * D:]),
        bt(b_gru_hh[0:D]), bt(b_gru_hh[D:2 * D]), bt(b_gru_hh[2 * D:]),
    )

    ehp = _tc_ehp(edge_attr.T, W_e1, b_e1)
    hp = _tc_project(x, W_proj, b_proj).reshape(NP, 128)
    hiddenp = hp
    for step in range(STEPS):
        h_src = _sc_gather(hp.reshape(N, D), src3)
        msg_p = _tc_message(ehp, h_src.reshape(E // 16, 128),
                            BD2, b2t, G)
        agg2 = _sc_scatter_add(msg_p.reshape(E, D), dst3, zeros_nd)
        if step < STEPS - 1:
            hiddenp = _tc_gru(agg2.reshape(NC, NP, 128), hp, hiddenp, kron_w)
            hp = hiddenp
        else:
            return _tc_gru_readout(
                agg2.reshape(NC, NP, 128), hp, hiddenp, kron_w,
                batch.reshape(NP, 16), W_r1, b_r1, W_r2, b_r2, W_p, b_p)
